# Initial kernel scaffold; baseline (speedup 1.0000x reference)
import functools

import jax
import jax.numpy as jnp
from jax import lax
from jax.experimental import pallas as pl
from jax.experimental.pallas import tpu as pltpu
from jax.experimental.pallas import tpu_sc as plsc

N_MOV = 600000
N_PHASE = 400000
N_INT = 50000
D = 128
H = 256

_NC = 2
_NS = 16
_NW = _NC * _NS

_EW = 147
_PW = 98
_E_PAD = _NW * _EW * 128
_P_PAD = _NW * _PW * 128
_SEG_PAD = 50176
_SEG_TILE = _SEG_PAD // _NS
_SEG_W = _SEG_PAD // _NW
_DUMMY_SEG = 50100

_MLP_ROWS = 1600


def _mlp_block(x_ref, w1_ref, b1_ref, w2t_ref, b2_ref, o_ref):
    x = x_ref[...]
    h = jnp.dot(x, w1_ref[...], preferred_element_type=jnp.float32) + b1_ref[...]
    h = jnp.maximum(h, 0.0)
    o_ref[...] = jnp.sum(h * w2t_ref[...], axis=1, keepdims=True) + b2_ref[...]


def _mlp(x, W1, b1, W2, b2):
    n = x.shape[0]
    return pl.pallas_call(
        _mlp_block,
        grid=(n // _MLP_ROWS,),
        in_specs=[
            pl.BlockSpec((_MLP_ROWS, D), lambda i: (i, 0)),
            pl.BlockSpec((D, H), lambda i: (0, 0)),
            pl.BlockSpec((1, H), lambda i: (0, 0)),
            pl.BlockSpec((1, H), lambda i: (0, 0)),
            pl.BlockSpec((1, 1), lambda i: (0, 0)),
        ],
        out_specs=pl.BlockSpec((_MLP_ROWS, 1), lambda i: (i, 0)),
        out_shape=jax.ShapeDtypeStruct((n, 1), jnp.float32),
    )(x, W1, b1.reshape(1, H), W2.reshape(1, H), b2.reshape(1, 1))


def _sc_mesh():
    return plsc.VectorSubcoreMesh(
        core_axis_name="c", subcore_axis_name="s",
        num_cores=_NC, num_subcores=_NS)


def _sc_accumulate(vmov, msrc2, mdst2, pi2, adv2):
    out_type = (
        jax.ShapeDtypeStruct((_NC, _SEG_PAD), jnp.float32),
        jax.ShapeDtypeStruct((_NC, _SEG_PAD), jnp.float32),
        jax.ShapeDtypeStruct((_NC, _SEG_PAD), jnp.float32),
    )

    @functools.partial(
        pl.kernel, out_type=out_type, mesh=_sc_mesh(),
        scratch_types=[
            pltpu.VMEM((_EW, 128), jnp.int32),
            pltpu.VMEM((_EW, 128), jnp.int32),
            pltpu.VMEM((_EW, 128), jnp.float32),
            pltpu.VMEM((128,), jnp.float32),
            pltpu.VMEM((_SEG_TILE,), jnp.float32),
            pltpu.VMEM_SHARED((_SEG_PAD,), jnp.float32),
            pltpu.VMEM_SHARED((_SEG_PAD,), jnp.float32),
            pltpu.VMEM_SHARED((_SEG_PAD,), jnp.float32),
            pltpu.SemaphoreType.DMA,
        ])
    def k(vmov_h, msrc_h, mdst_h, pi_h, adv_h, svp_h, psp_h, pcp_h,
          bidx, bdst, bval, bones, bzero, acc_sv, acc_ps, acc_pc, sem):
        c = lax.axis_index("c")
        s = lax.axis_index("s")
        w = s * _NC + c

        def zfill(i, carry):
            bzero[pl.ds(i * 16, 16)] = jnp.zeros((16,), jnp.float32)
            return carry
        lax.fori_loop(0, _SEG_TILE // 16, zfill, 0)
        for kk in range(8):
            bones[pl.ds(kk * 16, 16)] = jnp.ones((16,), jnp.float32)

        tile_sl = pl.ds(s * _SEG_TILE, _SEG_TILE)
        pltpu.sync_copy(bzero, acc_sv.at[tile_sl])
        pltpu.sync_copy(bzero, acc_ps.at[tile_sl])
        pltpu.sync_copy(bzero, acc_pc.at[tile_sl])
        plsc.subcore_barrier()

        pltpu.sync_copy(msrc_h.at[pl.ds(w * _EW, _EW)], bidx)
        pltpu.sync_copy(mdst_h.at[pl.ds(w * _EW, _EW)], bdst)

        def estep(j, carry):
            pltpu.async_copy(vmov_h.at[bidx.at[j]], bval.at[j], sem).wait()
            pltpu.sync_copy(bval.at[j], acc_sv.at[bdst.at[j]], add=True)
            return carry
        lax.fori_loop(0, _EW, estep, 0)

        pltpu.sync_copy(pi_h.at[pl.ds(w * _PW, _PW)], bdst.at[pl.ds(0, _PW)])
        pltpu.sync_copy(adv_h.at[pl.ds(w * _PW, _PW)], bval.at[pl.ds(0, _PW)])

        def pstep(j, carry):
            pltpu.sync_copy(bval.at[j], acc_ps.at[bdst.at[j]], add=True)
            pltpu.sync_copy(bones, acc_pc.at[bdst.at[j]], add=True)
            return carry
        lax.fori_loop(0, _PW, pstep, 0)

        plsc.subcore_barrier()
        pltpu.sync_copy(acc_sv.at[tile_sl], svp_h.at[c, tile_sl])
        pltpu.sync_copy(acc_ps.at[tile_sl], psp_h.at[c, tile_sl])
        pltpu.sync_copy(acc_pc.at[tile_sl], pcp_h.at[c, tile_sl])

    return k(vmov, msrc2, mdst2, pi2, adv2)


def _sc_combine(svp, psp, pcp):
    @functools.partial(
        pl.kernel, out_type=jax.ShapeDtypeStruct((_SEG_PAD,), jnp.float32),
        mesh=_sc_mesh(),
        scratch_types=[pltpu.VMEM((_SEG_W,), jnp.float32) for _ in range(7)])
    def k(svp_h, psp_h, pcp_h, t_h, a0, a1, p0, p1, c0, c1, tb):
        c = lax.axis_index("c")
        s = lax.axis_index("s")
        w = s * _NC + c
        base = w * _SEG_W
        pltpu.sync_copy(svp_h.at[0, pl.ds(base, _SEG_W)], a0)
        pltpu.sync_copy(svp_h.at[1, pl.ds(base, _SEG_W)], a1)
        pltpu.sync_copy(psp_h.at[0, pl.ds(base, _SEG_W)], p0)
        pltpu.sync_copy(psp_h.at[1, pl.ds(base, _SEG_W)], p1)
        pltpu.sync_copy(pcp_h.at[0, pl.ds(base, _SEG_W)], c0)
        pltpu.sync_copy(pcp_h.at[1, pl.ds(base, _SEG_W)], c1)

        def step(i, carry):
            sl = pl.ds(i * 16, 16)
            sv = a0[sl] + a1[sl]
            ps = p0[sl] + p1[sl]
            pc = c0[sl] + c1[sl]
            tb[sl] = sv - ps / jnp.maximum(pc, 1.0)
            return carry
        lax.fori_loop(0, _SEG_W // 16, step, 0)
        pltpu.sync_copy(tb, t_h.at[pl.ds(base, _SEG_W)])

    return k(svp, psp, pcp)


def _sc_gather_out(t, pi1, adv1):
    chunk = _PW * 128

    @functools.partial(
        pl.kernel, out_type=jax.ShapeDtypeStruct((_P_PAD,), jnp.float32),
        mesh=_sc_mesh(),
        scratch_types=[
            pltpu.VMEM((chunk,), jnp.int32),
            pltpu.VMEM((chunk,), jnp.float32),
            pltpu.VMEM((chunk,), jnp.float32),
            pltpu.SemaphoreType.DMA,
        ])
    def k(t_h, pi_h, adv_h, out_h, bpi, badv, btv, sem):
        c = lax.axis_index("c")
        s = lax.axis_index("s")
        w = s * _NC + c
        base = w * chunk
        pltpu.sync_copy(pi_h.at[pl.ds(base, chunk)], bpi)
        pltpu.sync_copy(adv_h.at[pl.ds(base, chunk)], badv)

        def gstep(j, carry):
            sl = pl.ds(j * 128, 128)
            pltpu.async_copy(t_h.at[bpi.at[sl]], btv.at[sl], sem).wait()
            return carry
        lax.fori_loop(0, _PW, gstep, 0)

        def astep(i, carry):
            sl = pl.ds(i * 16, 16)
            badv[sl] = badv[sl] + btv[sl]
            return carry
        lax.fori_loop(0, chunk // 16, astep, 0)
        pltpu.sync_copy(badv, out_h.at[pl.ds(base, chunk)])

    return k(t, pi1, adv1)


def kernel(x_movement, x_phase, mi_src, mi_dst, pi_dst, W1, b1, W2, b2):
    v_mov = _mlp(x_movement, W1, b1, W2, b2)[:, 0]
    adv = _mlp(x_phase, W1, b1, W2, b2)[:, 0]

    e_pad = _E_PAD - N_MOV
    p_pad = _P_PAD - N_PHASE
    msrc2 = jnp.concatenate(
        [mi_src, jnp.zeros((e_pad,), jnp.int32)]).reshape(-1, 128)
    mdst2 = jnp.concatenate(
        [mi_dst, jnp.full((e_pad,), _DUMMY_SEG, jnp.int32)]).reshape(-1, 128)
    pi_p = jnp.concatenate([pi_dst, jnp.full((p_pad,), _DUMMY_SEG, jnp.int32)])
    adv_p = jnp.concatenate([adv, jnp.zeros((p_pad,), jnp.float32)])

    svp, psp, pcp = _sc_accumulate(
        v_mov, msrc2, mdst2, pi_p.reshape(-1, 128), adv_p.reshape(-1, 128))
    t = _sc_combine(svp, psp, pcp)
    out_p = _sc_gather_out(t, pi_p, adv_p)
    return out_p[:N_PHASE], pi_dst


# speedup vs baseline: 14.6723x; 14.6723x over previous
import functools

import jax
import jax.numpy as jnp
from jax import lax
from jax.experimental import pallas as pl
from jax.experimental.pallas import tpu as pltpu
from jax.experimental.pallas import tpu_sc as plsc

N_MOV = 600000
N_PHASE = 400000
N_INT = 50000
D = 128
H = 256

_NC = 2
_NS = 16
_NW = _NC * _NS

_EW = 152
_PW = 104
_E_PAD = _NW * _EW * 128
_P_PAD = _NW * _PW * 128
_SEG_PAD = 50176
_SEG_TILE = _SEG_PAD // _NS
_SEG_W = _SEG_PAD // _NW
_DUMMY_SEG = 50100

_MLP_ROWS = 1600


def _mlp_block(x_ref, w1_ref, b1_ref, w2t_ref, b2_ref, o_ref):
    x = x_ref[...]
    h = jnp.dot(x, w1_ref[...], preferred_element_type=jnp.float32) + b1_ref[...]
    h = jnp.maximum(h, 0.0)
    o_ref[...] = jnp.sum(h * w2t_ref[...], axis=1, keepdims=True) + b2_ref[...]


def _mlp(x, W1, b1, W2, b2):
    n = x.shape[0]
    return pl.pallas_call(
        _mlp_block,
        grid=(n // _MLP_ROWS,),
        in_specs=[
            pl.BlockSpec((_MLP_ROWS, D), lambda i: (i, 0)),
            pl.BlockSpec((D, H), lambda i: (0, 0)),
            pl.BlockSpec((1, H), lambda i: (0, 0)),
            pl.BlockSpec((1, H), lambda i: (0, 0)),
            pl.BlockSpec((1, 1), lambda i: (0, 0)),
        ],
        out_specs=pl.BlockSpec((_MLP_ROWS, 1), lambda i: (i, 0)),
        out_shape=jax.ShapeDtypeStruct((n, 1), jnp.float32),
    )(x, W1, b1.reshape(1, H), W2.reshape(1, H), b2.reshape(1, 1))


def _sc_mesh():
    return plsc.VectorSubcoreMesh(
        core_axis_name="c", subcore_axis_name="s",
        num_cores=_NC, num_subcores=_NS)


def _sc_accumulate(vmov, msrc2, mdst2, pi2, adv2):
    out_type = (
        jax.ShapeDtypeStruct((_NC * _SEG_PAD,), jnp.float32),
        jax.ShapeDtypeStruct((_NC * _SEG_PAD,), jnp.float32),
        jax.ShapeDtypeStruct((_NC * _SEG_PAD,), jnp.float32),
    )

    @functools.partial(
        pl.kernel, out_type=out_type, mesh=_sc_mesh(),
        scratch_types=[
            pltpu.VMEM((_EW, 128), jnp.int32),
            pltpu.VMEM((_EW, 128), jnp.int32),
            pltpu.VMEM((_EW, 128), jnp.float32),
            pltpu.VMEM((128,), jnp.float32),
            pltpu.VMEM((_SEG_TILE,), jnp.float32),
            pltpu.VMEM_SHARED((_SEG_PAD,), jnp.float32),
            pltpu.VMEM_SHARED((_SEG_PAD,), jnp.float32),
            pltpu.VMEM_SHARED((_SEG_PAD,), jnp.float32),
            pltpu.SemaphoreType.DMA,
        ])
    def k(vmov_h, msrc_h, mdst_h, pi_h, adv_h, svp_h, psp_h, pcp_h,
          bidx, bdst, bval, bones, bzero, acc_sv, acc_ps, acc_pc, sem):
        c = lax.axis_index("c")
        s = lax.axis_index("s")
        w = s * _NC + c

        def zfill(i, carry):
            bzero[pl.ds(i * 16, 16)] = jnp.zeros((16,), jnp.float32)
            return carry
        lax.fori_loop(0, _SEG_TILE // 16, zfill, 0)
        for kk in range(8):
            bones[pl.ds(kk * 16, 16)] = jnp.ones((16,), jnp.float32)

        tile_sl = pl.ds(s * _SEG_TILE, _SEG_TILE)
        pltpu.sync_copy(bzero, acc_sv.at[tile_sl])
        pltpu.sync_copy(bzero, acc_ps.at[tile_sl])
        pltpu.sync_copy(bzero, acc_pc.at[tile_sl])
        plsc.subcore_barrier()

        pltpu.sync_copy(msrc_h.at[pl.ds(w * _EW, _EW)], bidx)
        pltpu.sync_copy(mdst_h.at[pl.ds(w * _EW, _EW)], bdst)

        def estep(j, carry):
            pltpu.async_copy(vmov_h.at[bidx.at[j]], bval.at[j], sem).wait()
            pltpu.sync_copy(bval.at[j], acc_sv.at[bdst.at[j]], add=True)
            return carry
        lax.fori_loop(0, _EW, estep, 0)

        pltpu.sync_copy(pi_h.at[pl.ds(w * _PW, _PW)], bdst.at[pl.ds(0, _PW)])
        pltpu.sync_copy(adv_h.at[pl.ds(w * _PW, _PW)], bval.at[pl.ds(0, _PW)])

        def pstep(j, carry):
            pltpu.sync_copy(bval.at[j], acc_ps.at[bdst.at[j]], add=True)
            pltpu.sync_copy(bones, acc_pc.at[bdst.at[j]], add=True)
            return carry
        lax.fori_loop(0, _PW, pstep, 0)

        plsc.subcore_barrier()
        out_sl = pl.ds(c * _SEG_PAD + s * _SEG_TILE, _SEG_TILE)
        pltpu.sync_copy(acc_sv.at[tile_sl], bzero)
        pltpu.sync_copy(bzero, svp_h.at[out_sl])
        pltpu.sync_copy(acc_ps.at[tile_sl], bzero)
        pltpu.sync_copy(bzero, psp_h.at[out_sl])
        pltpu.sync_copy(acc_pc.at[tile_sl], bzero)
        pltpu.sync_copy(bzero, pcp_h.at[out_sl])

    return k(vmov, msrc2, mdst2, pi2, adv2)


def _sc_combine(svp, psp, pcp):
    @functools.partial(
        pl.kernel, out_type=jax.ShapeDtypeStruct((_SEG_PAD,), jnp.float32),
        mesh=_sc_mesh(),
        scratch_types=[pltpu.VMEM((_SEG_W,), jnp.float32) for _ in range(7)])
    def k(svp_h, psp_h, pcp_h, t_h, a0, a1, p0, p1, c0, c1, tb):
        c = lax.axis_index("c")
        s = lax.axis_index("s")
        w = s * _NC + c
        base = w * _SEG_W
        pltpu.sync_copy(svp_h.at[pl.ds(base, _SEG_W)], a0)
        pltpu.sync_copy(svp_h.at[pl.ds(_SEG_PAD + base, _SEG_W)], a1)
        pltpu.sync_copy(psp_h.at[pl.ds(base, _SEG_W)], p0)
        pltpu.sync_copy(psp_h.at[pl.ds(_SEG_PAD + base, _SEG_W)], p1)
        pltpu.sync_copy(pcp_h.at[pl.ds(base, _SEG_W)], c0)
        pltpu.sync_copy(pcp_h.at[pl.ds(_SEG_PAD + base, _SEG_W)], c1)

        def step(i, carry):
            sl = pl.ds(i * 16, 16)
            sv = a0[sl] + a1[sl]
            ps = p0[sl] + p1[sl]
            pc = c0[sl] + c1[sl]
            tb[sl] = sv - ps / jnp.maximum(pc, 1.0)
            return carry
        lax.fori_loop(0, _SEG_W // 16, step, 0)
        pltpu.sync_copy(tb, t_h.at[pl.ds(base, _SEG_W)])

    return k(svp, psp, pcp)


def _sc_gather_out(t, pi1, adv1):
    chunk = _PW * 128

    @functools.partial(
        pl.kernel, out_type=jax.ShapeDtypeStruct((_P_PAD,), jnp.float32),
        mesh=_sc_mesh(),
        scratch_types=[
            pltpu.VMEM((chunk,), jnp.int32),
            pltpu.VMEM((chunk,), jnp.float32),
            pltpu.VMEM((chunk,), jnp.float32),
            pltpu.SemaphoreType.DMA,
        ])
    def k(t_h, pi_h, adv_h, out_h, bpi, badv, btv, sem):
        c = lax.axis_index("c")
        s = lax.axis_index("s")
        w = s * _NC + c
        base = w * chunk
        pltpu.sync_copy(pi_h.at[pl.ds(base, chunk)], bpi)
        pltpu.sync_copy(adv_h.at[pl.ds(base, chunk)], badv)

        def gstep(j, carry):
            sl = pl.ds(j * 128, 128)
            pltpu.async_copy(t_h.at[bpi.at[sl]], btv.at[sl], sem).wait()
            return carry
        lax.fori_loop(0, _PW, gstep, 0)

        def astep(i, carry):
            sl = pl.ds(i * 16, 16)
            badv[sl] = badv[sl] + btv[sl]
            return carry
        lax.fori_loop(0, chunk // 16, astep, 0)
        pltpu.sync_copy(badv, out_h.at[pl.ds(base, chunk)])

    return k(t, pi1, adv1)


def kernel(x_movement, x_phase, mi_src, mi_dst, pi_dst, W1, b1, W2, b2):
    v_mov = _mlp(x_movement, W1, b1, W2, b2)[:, 0]
    adv = _mlp(x_phase, W1, b1, W2, b2)[:, 0]

    e_pad = _E_PAD - N_MOV
    p_pad = _P_PAD - N_PHASE
    msrc2 = jnp.concatenate(
        [mi_src, jnp.zeros((e_pad,), jnp.int32)]).reshape(-1, 128)
    mdst2 = jnp.concatenate(
        [mi_dst, jnp.full((e_pad,), _DUMMY_SEG, jnp.int32)]).reshape(-1, 128)
    pi_p = jnp.concatenate([pi_dst, jnp.full((p_pad,), _DUMMY_SEG, jnp.int32)])
    adv_p = jnp.concatenate([adv, jnp.zeros((p_pad,), jnp.float32)])

    svp, psp, pcp = _sc_accumulate(
        v_mov, msrc2, mdst2, pi_p.reshape(-1, 128), adv_p.reshape(-1, 128))
    t = _sc_combine(svp, psp, pcp)
    out_p = _sc_gather_out(t, pi_p, adv_p)
    return out_p[:N_PHASE], pi_dst


# speedup vs baseline: 18.1891x; 1.2397x over previous
import functools

import jax
import jax.numpy as jnp
from jax import lax
from jax.experimental import pallas as pl
from jax.experimental.pallas import tpu as pltpu
from jax.experimental.pallas import tpu_sc as plsc

N_MOV = 600000
N_PHASE = 400000
N_INT = 50000
D = 128
H = 256

_NC = 2
_NS = 16
_NW = _NC * _NS

_EW = 152
_PW = 104
_E_PAD = _NW * _EW * 128
_P_PAD = _NW * _PW * 128
_SEG_PAD = 50176
_SEG_TILE = _SEG_PAD // _NS
_SEG_W = _SEG_PAD // _NW
_DUMMY_SEG = 50100

_MLP_ROWS = 4000


def _mlp_block(x_ref, w1_ref, b1_ref, w2t_ref, b2_ref, o_ref):
    x = x_ref[...].astype(jnp.bfloat16)
    w1 = w1_ref[...].astype(jnp.bfloat16)
    h = jnp.dot(x, w1, preferred_element_type=jnp.float32) + b1_ref[...]
    h = jnp.maximum(h, 0.0)
    o_ref[...] = jnp.sum(h * w2t_ref[...], axis=1, keepdims=True) + b2_ref[...]


def _mlp(x, W1, b1, W2, b2):
    n = x.shape[0]
    return pl.pallas_call(
        _mlp_block,
        grid=(n // _MLP_ROWS,),
        in_specs=[
            pl.BlockSpec((_MLP_ROWS, D), lambda i: (i, 0)),
            pl.BlockSpec((D, H), lambda i: (0, 0)),
            pl.BlockSpec((1, H), lambda i: (0, 0)),
            pl.BlockSpec((1, H), lambda i: (0, 0)),
            pl.BlockSpec((1, 1), lambda i: (0, 0)),
        ],
        out_specs=pl.BlockSpec((_MLP_ROWS, 1), lambda i: (i, 0)),
        out_shape=jax.ShapeDtypeStruct((n, 1), jnp.float32),
    )(x, W1, b1.reshape(1, H), W2.reshape(1, H), b2.reshape(1, 1))


def _sc_mesh():
    return plsc.VectorSubcoreMesh(
        core_axis_name="c", subcore_axis_name="s",
        num_cores=_NC, num_subcores=_NS)


def _sc_accumulate(vmov, msrc2, mdst2, pi2, adv2):
    out_type = (
        jax.ShapeDtypeStruct((_NC * _SEG_PAD,), jnp.float32),
        jax.ShapeDtypeStruct((_NC * _SEG_PAD,), jnp.float32),
        jax.ShapeDtypeStruct((_NC * _SEG_PAD,), jnp.float32),
    )

    @functools.partial(
        pl.kernel, out_type=out_type, mesh=_sc_mesh(),
        scratch_types=[
            pltpu.VMEM((_EW, 128), jnp.int32),
            pltpu.VMEM((_EW, 128), jnp.int32),
            pltpu.VMEM((_EW, 128), jnp.float32),
            pltpu.VMEM((128,), jnp.float32),
            pltpu.VMEM((_SEG_TILE,), jnp.float32),
            pltpu.VMEM_SHARED((_SEG_PAD,), jnp.float32),
            pltpu.VMEM_SHARED((_SEG_PAD,), jnp.float32),
            pltpu.VMEM_SHARED((_SEG_PAD,), jnp.float32),
            pltpu.SemaphoreType.DMA,
        ])
    def k(vmov_h, msrc_h, mdst_h, pi_h, adv_h, svp_h, psp_h, pcp_h,
          bidx, bdst, bval, bones, bzero, acc_sv, acc_ps, acc_pc, sem):
        c = lax.axis_index("c")
        s = lax.axis_index("s")
        w = s * _NC + c

        def zfill(i, carry):
            bzero[pl.ds(i * 16, 16)] = jnp.zeros((16,), jnp.float32)
            return carry
        lax.fori_loop(0, _SEG_TILE // 16, zfill, 0)
        for kk in range(8):
            bones[pl.ds(kk * 16, 16)] = jnp.ones((16,), jnp.float32)

        tile_sl = pl.ds(s * _SEG_TILE, _SEG_TILE)
        pltpu.sync_copy(bzero, acc_sv.at[tile_sl])
        pltpu.sync_copy(bzero, acc_ps.at[tile_sl])
        pltpu.sync_copy(bzero, acc_pc.at[tile_sl])
        plsc.subcore_barrier()

        pltpu.sync_copy(msrc_h.at[pl.ds(w * _EW, _EW)], bidx)
        pltpu.sync_copy(mdst_h.at[pl.ds(w * _EW, _EW)], bdst)

        def estep(j, carry):
            pltpu.async_copy(vmov_h.at[bidx.at[j]], bval.at[j], sem).wait()
            pltpu.sync_copy(bval.at[j], acc_sv.at[bdst.at[j]], add=True)
            return carry
        lax.fori_loop(0, _EW, estep, 0)

        pltpu.sync_copy(pi_h.at[pl.ds(w * _PW, _PW)], bdst.at[pl.ds(0, _PW)])
        pltpu.sync_copy(adv_h.at[pl.ds(w * _PW, _PW)], bval.at[pl.ds(0, _PW)])

        def pstep(j, carry):
            pltpu.sync_copy(bval.at[j], acc_ps.at[bdst.at[j]], add=True)
            pltpu.sync_copy(bones, acc_pc.at[bdst.at[j]], add=True)
            return carry
        lax.fori_loop(0, _PW, pstep, 0)

        plsc.subcore_barrier()
        out_sl = pl.ds(c * _SEG_PAD + s * _SEG_TILE, _SEG_TILE)
        pltpu.sync_copy(acc_sv.at[tile_sl], bzero)
        pltpu.sync_copy(bzero, svp_h.at[out_sl])
        pltpu.sync_copy(acc_ps.at[tile_sl], bzero)
        pltpu.sync_copy(bzero, psp_h.at[out_sl])
        pltpu.sync_copy(acc_pc.at[tile_sl], bzero)
        pltpu.sync_copy(bzero, pcp_h.at[out_sl])

    return k(vmov, msrc2, mdst2, pi2, adv2)


def _sc_combine(svp, psp, pcp):
    @functools.partial(
        pl.kernel, out_type=jax.ShapeDtypeStruct((_SEG_PAD,), jnp.float32),
        mesh=_sc_mesh(),
        scratch_types=[pltpu.VMEM((_SEG_W,), jnp.float32) for _ in range(7)])
    def k(svp_h, psp_h, pcp_h, t_h, a0, a1, p0, p1, c0, c1, tb):
        c = lax.axis_index("c")
        s = lax.axis_index("s")
        w = s * _NC + c
        base = w * _SEG_W
        pltpu.sync_copy(svp_h.at[pl.ds(base, _SEG_W)], a0)
        pltpu.sync_copy(svp_h.at[pl.ds(_SEG_PAD + base, _SEG_W)], a1)
        pltpu.sync_copy(psp_h.at[pl.ds(base, _SEG_W)], p0)
        pltpu.sync_copy(psp_h.at[pl.ds(_SEG_PAD + base, _SEG_W)], p1)
        pltpu.sync_copy(pcp_h.at[pl.ds(base, _SEG_W)], c0)
        pltpu.sync_copy(pcp_h.at[pl.ds(_SEG_PAD + base, _SEG_W)], c1)

        def step(i, carry):
            sl = pl.ds(i * 16, 16)
            sv = a0[sl] + a1[sl]
            ps = p0[sl] + p1[sl]
            pc = c0[sl] + c1[sl]
            tb[sl] = sv - ps / jnp.maximum(pc, 1.0)
            return carry
        lax.fori_loop(0, _SEG_W // 16, step, 0)
        pltpu.sync_copy(tb, t_h.at[pl.ds(base, _SEG_W)])

    return k(svp, psp, pcp)


def _sc_gather_out(t, pi1, adv1):
    chunk = _PW * 128

    @functools.partial(
        pl.kernel, out_type=jax.ShapeDtypeStruct((_P_PAD,), jnp.float32),
        mesh=_sc_mesh(),
        scratch_types=[
            pltpu.VMEM((chunk,), jnp.int32),
            pltpu.VMEM((chunk,), jnp.float32),
            pltpu.VMEM((chunk,), jnp.float32),
            pltpu.SemaphoreType.DMA,
        ])
    def k(t_h, pi_h, adv_h, out_h, bpi, badv, btv, sem):
        c = lax.axis_index("c")
        s = lax.axis_index("s")
        w = s * _NC + c
        base = w * chunk
        pltpu.sync_copy(pi_h.at[pl.ds(base, chunk)], bpi)
        pltpu.sync_copy(adv_h.at[pl.ds(base, chunk)], badv)

        def gstep(j, carry):
            sl = pl.ds(j * 128, 128)
            pltpu.async_copy(t_h.at[bpi.at[sl]], btv.at[sl], sem).wait()
            return carry
        lax.fori_loop(0, _PW, gstep, 0)

        def astep(i, carry):
            sl = pl.ds(i * 16, 16)
            badv[sl] = badv[sl] + btv[sl]
            return carry
        lax.fori_loop(0, chunk // 16, astep, 0)
        pltpu.sync_copy(badv, out_h.at[pl.ds(base, chunk)])

    return k(t, pi1, adv1)


def kernel(x_movement, x_phase, mi_src, mi_dst, pi_dst, W1, b1, W2, b2):
    v_mov = _mlp(x_movement, W1, b1, W2, b2)[:, 0]
    adv = _mlp(x_phase, W1, b1, W2, b2)[:, 0]

    e_pad = _E_PAD - N_MOV
    p_pad = _P_PAD - N_PHASE
    msrc2 = jnp.concatenate(
        [mi_src, jnp.zeros((e_pad,), jnp.int32)]).reshape(-1, 128)
    mdst2 = jnp.concatenate(
        [mi_dst, jnp.full((e_pad,), _DUMMY_SEG, jnp.int32)]).reshape(-1, 128)
    pi_p = jnp.concatenate([pi_dst, jnp.full((p_pad,), _DUMMY_SEG, jnp.int32)])
    adv_p = jnp.concatenate([adv, jnp.zeros((p_pad,), jnp.float32)])

    svp, psp, pcp = _sc_accumulate(
        v_mov, msrc2, mdst2, pi_p.reshape(-1, 128), adv_p.reshape(-1, 128))
    t = _sc_combine(svp, psp, pcp)
    out_p = _sc_gather_out(t, pi_p, adv_p)
    return out_p[:N_PHASE], pi_dst


# speedup vs baseline: 24.6543x; 1.3554x over previous
import functools

import jax
import jax.numpy as jnp
from jax import lax
from jax.experimental import pallas as pl
from jax.experimental.pallas import tpu as pltpu
from jax.experimental.pallas import tpu_sc as plsc

N_MOV = 600000
N_PHASE = 400000
N_INT = 50000
D = 128
H = 256

_NC = 2
_NS = 16
_NW = _NC * _NS

_EW = 152
_PW = 104
_E_PAD = _NW * _EW * 128
_P_PAD = _NW * _PW * 128
_SEG_PAD = 50176
_SEG_TILE = _SEG_PAD // _NS
_SEG_W = _SEG_PAD // _NW
_DUMMY_SEG = 50100

_MLP_ROWS = 4096


def _mlp_block(x_ref, w1_ref, b1_ref, w2_ref, b2_ref, o_ref):
    x = x_ref[...].astype(jnp.bfloat16)
    w1 = w1_ref[...].astype(jnp.bfloat16)
    hT = lax.dot_general(w1, x, dimension_numbers=(((0,), (1,)), ((), ())),
                         preferred_element_type=jnp.float32)
    hT = jnp.maximum(hT + b1_ref[...], 0.0)
    o_ref[...] = jnp.sum(hT * w2_ref[...], axis=0) + b2_ref[0, 0]


def _mlp(x, W1, b1, W2, b2):
    n = x.shape[0]
    return pl.pallas_call(
        _mlp_block,
        grid=((n + _MLP_ROWS - 1) // _MLP_ROWS,),
        in_specs=[
            pl.BlockSpec((_MLP_ROWS, D), lambda i: (i, 0)),
            pl.BlockSpec((D, H), lambda i: (0, 0)),
            pl.BlockSpec((H, 1), lambda i: (0, 0)),
            pl.BlockSpec((H, 1), lambda i: (0, 0)),
            pl.BlockSpec((1, 1), lambda i: (0, 0)),
        ],
        out_specs=pl.BlockSpec((_MLP_ROWS,), lambda i: (i,)),
        out_shape=jax.ShapeDtypeStruct((n,), jnp.float32),
    )(x, W1, b1.reshape(H, 1), W2, b2.reshape(1, 1))


def _sc_mesh():
    return plsc.VectorSubcoreMesh(
        core_axis_name="c", subcore_axis_name="s",
        num_cores=_NC, num_subcores=_NS)


def _sc_accumulate(vmov, msrc2, mdst2, pi2, adv2):
    out_type = (
        jax.ShapeDtypeStruct((_NC * _SEG_PAD,), jnp.float32),
        jax.ShapeDtypeStruct((_NC * _SEG_PAD,), jnp.float32),
        jax.ShapeDtypeStruct((_NC * _SEG_PAD,), jnp.float32),
    )

    @functools.partial(
        pl.kernel, out_type=out_type, mesh=_sc_mesh(),
        scratch_types=[
            pltpu.VMEM((_EW, 128), jnp.int32),
            pltpu.VMEM((_EW, 128), jnp.int32),
            pltpu.VMEM((_EW, 128), jnp.float32),
            pltpu.VMEM((128,), jnp.float32),
            pltpu.VMEM((_SEG_TILE,), jnp.float32),
            pltpu.VMEM_SHARED((_SEG_PAD,), jnp.float32),
            pltpu.VMEM_SHARED((_SEG_PAD,), jnp.float32),
            pltpu.VMEM_SHARED((_SEG_PAD,), jnp.float32),
            pltpu.SemaphoreType.DMA,
        ])
    def k(vmov_h, msrc_h, mdst_h, pi_h, adv_h, svp_h, psp_h, pcp_h,
          bidx, bdst, bval, bones, bzero, acc_sv, acc_ps, acc_pc, sem):
        c = lax.axis_index("c")
        s = lax.axis_index("s")
        w = s * _NC + c

        def zfill(i, carry):
            bzero[pl.ds(i * 16, 16)] = jnp.zeros((16,), jnp.float32)
            return carry
        lax.fori_loop(0, _SEG_TILE // 16, zfill, 0)
        for kk in range(8):
            bones[pl.ds(kk * 16, 16)] = jnp.ones((16,), jnp.float32)

        tile_sl = pl.ds(s * _SEG_TILE, _SEG_TILE)
        pltpu.sync_copy(bzero, acc_sv.at[tile_sl])
        pltpu.sync_copy(bzero, acc_ps.at[tile_sl])
        pltpu.sync_copy(bzero, acc_pc.at[tile_sl])
        plsc.subcore_barrier()

        pltpu.sync_copy(msrc_h.at[pl.ds(w * _EW, _EW)], bidx)
        pltpu.sync_copy(mdst_h.at[pl.ds(w * _EW, _EW)], bdst)

        def estep(j, carry):
            pltpu.async_copy(vmov_h.at[bidx.at[j]], bval.at[j], sem).wait()
            pltpu.sync_copy(bval.at[j], acc_sv.at[bdst.at[j]], add=True)
            return carry
        lax.fori_loop(0, _EW, estep, 0)

        pltpu.sync_copy(pi_h.at[pl.ds(w * _PW, _PW)], bdst.at[pl.ds(0, _PW)])
        pltpu.sync_copy(adv_h.at[pl.ds(w * _PW, _PW)], bval.at[pl.ds(0, _PW)])

        def pstep(j, carry):
            pltpu.sync_copy(bval.at[j], acc_ps.at[bdst.at[j]], add=True)
            pltpu.sync_copy(bones, acc_pc.at[bdst.at[j]], add=True)
            return carry
        lax.fori_loop(0, _PW, pstep, 0)

        plsc.subcore_barrier()
        out_sl = pl.ds(c * _SEG_PAD + s * _SEG_TILE, _SEG_TILE)
        pltpu.sync_copy(acc_sv.at[tile_sl], bzero)
        pltpu.sync_copy(bzero, svp_h.at[out_sl])
        pltpu.sync_copy(acc_ps.at[tile_sl], bzero)
        pltpu.sync_copy(bzero, psp_h.at[out_sl])
        pltpu.sync_copy(acc_pc.at[tile_sl], bzero)
        pltpu.sync_copy(bzero, pcp_h.at[out_sl])

    return k(vmov, msrc2, mdst2, pi2, adv2)


def _sc_combine(svp, psp, pcp):
    @functools.partial(
        pl.kernel, out_type=jax.ShapeDtypeStruct((_SEG_PAD,), jnp.float32),
        mesh=_sc_mesh(),
        scratch_types=[pltpu.VMEM((_SEG_W,), jnp.float32) for _ in range(7)])
    def k(svp_h, psp_h, pcp_h, t_h, a0, a1, p0, p1, c0, c1, tb):
        c = lax.axis_index("c")
        s = lax.axis_index("s")
        w = s * _NC + c
        base = w * _SEG_W
        pltpu.sync_copy(svp_h.at[pl.ds(base, _SEG_W)], a0)
        pltpu.sync_copy(svp_h.at[pl.ds(_SEG_PAD + base, _SEG_W)], a1)
        pltpu.sync_copy(psp_h.at[pl.ds(base, _SEG_W)], p0)
        pltpu.sync_copy(psp_h.at[pl.ds(_SEG_PAD + base, _SEG_W)], p1)
        pltpu.sync_copy(pcp_h.at[pl.ds(base, _SEG_W)], c0)
        pltpu.sync_copy(pcp_h.at[pl.ds(_SEG_PAD + base, _SEG_W)], c1)

        def step(i, carry):
            sl = pl.ds(i * 16, 16)
            sv = a0[sl] + a1[sl]
            ps = p0[sl] + p1[sl]
            pc = c0[sl] + c1[sl]
            tb[sl] = sv - ps / jnp.maximum(pc, 1.0)
            return carry
        lax.fori_loop(0, _SEG_W // 16, step, 0)
        pltpu.sync_copy(tb, t_h.at[pl.ds(base, _SEG_W)])

    return k(svp, psp, pcp)


def _sc_gather_out(t, pi1, adv1):
    chunk = _PW * 128

    @functools.partial(
        pl.kernel, out_type=jax.ShapeDtypeStruct((_P_PAD,), jnp.float32),
        mesh=_sc_mesh(),
        scratch_types=[
            pltpu.VMEM((chunk,), jnp.int32),
            pltpu.VMEM((chunk,), jnp.float32),
            pltpu.VMEM((chunk,), jnp.float32),
            pltpu.SemaphoreType.DMA,
        ])
    def k(t_h, pi_h, adv_h, out_h, bpi, badv, btv, sem):
        c = lax.axis_index("c")
        s = lax.axis_index("s")
        w = s * _NC + c
        base = w * chunk
        pltpu.sync_copy(pi_h.at[pl.ds(base, chunk)], bpi)
        pltpu.sync_copy(adv_h.at[pl.ds(base, chunk)], badv)

        def gstep(j, carry):
            sl = pl.ds(j * 128, 128)
            pltpu.async_copy(t_h.at[bpi.at[sl]], btv.at[sl], sem).wait()
            return carry
        lax.fori_loop(0, _PW, gstep, 0)

        def astep(i, carry):
            sl = pl.ds(i * 16, 16)
            badv[sl] = badv[sl] + btv[sl]
            return carry
        lax.fori_loop(0, chunk // 16, astep, 0)
        pltpu.sync_copy(badv, out_h.at[pl.ds(base, chunk)])

    return k(t, pi1, adv1)


def kernel(x_movement, x_phase, mi_src, mi_dst, pi_dst, W1, b1, W2, b2):
    v_mov = _mlp(x_movement, W1, b1, W2, b2)
    adv = _mlp(x_phase, W1, b1, W2, b2)

    e_pad = _E_PAD - N_MOV
    p_pad = _P_PAD - N_PHASE
    msrc2 = jnp.concatenate(
        [mi_src, jnp.zeros((e_pad,), jnp.int32)]).reshape(-1, 128)
    mdst2 = jnp.concatenate(
        [mi_dst, jnp.full((e_pad,), _DUMMY_SEG, jnp.int32)]).reshape(-1, 128)
    pi_p = jnp.concatenate([pi_dst, jnp.full((p_pad,), _DUMMY_SEG, jnp.int32)])
    adv_p = jnp.concatenate([adv, jnp.zeros((p_pad,), jnp.float32)])

    svp, psp, pcp = _sc_accumulate(
        v_mov, msrc2, mdst2, pi_p.reshape(-1, 128), adv_p.reshape(-1, 128))
    t = _sc_combine(svp, psp, pcp)
    out_p = _sc_gather_out(t, pi_p, adv_p)
    return out_p[:N_PHASE], pi_dst


# speedup vs baseline: 25.4745x; 1.0333x over previous
import functools

import jax
import jax.numpy as jnp
from jax import lax
from jax.experimental import pallas as pl
from jax.experimental.pallas import tpu as pltpu
from jax.experimental.pallas import tpu_sc as plsc

N_MOV = 600000
N_PHASE = 400000
N_INT = 50000
D = 128
H = 256

_NC = 2
_NS = 16
_NW = _NC * _NS

_EW = 152
_PW = 104
_E_PAD = _NW * _EW * 128
_P_PAD = _NW * _PW * 128
_SEG_PAD = 50176
_SEG_TILE = _SEG_PAD // _NS
_SEG_W = _SEG_PAD // _NW
_DUMMY_SEG = 50100
_ECH = _EW * 128
_PCH = _PW * 128

_MLP_ROWS = 4096


def _mlp_block(x_ref, w1_ref, b1_ref, w2_ref, b2_ref, o_ref):
    x = x_ref[...].astype(jnp.bfloat16)
    w1 = w1_ref[...].astype(jnp.bfloat16)
    hT = lax.dot_general(w1, x, dimension_numbers=(((0,), (1,)), ((), ())),
                         preferred_element_type=jnp.float32)
    hT = jnp.maximum(hT + b1_ref[...], 0.0)
    o_ref[...] = jnp.sum(hT * w2_ref[...], axis=0) + b2_ref[0, 0]


def _mlp(x, W1, b1, W2, b2):
    n = x.shape[0]
    return pl.pallas_call(
        _mlp_block,
        grid=((n + _MLP_ROWS - 1) // _MLP_ROWS,),
        in_specs=[
            pl.BlockSpec((_MLP_ROWS, D), lambda i: (i, 0)),
            pl.BlockSpec((D, H), lambda i: (0, 0)),
            pl.BlockSpec((H, 1), lambda i: (0, 0)),
            pl.BlockSpec((H, 1), lambda i: (0, 0)),
            pl.BlockSpec((1, 1), lambda i: (0, 0)),
        ],
        out_specs=pl.BlockSpec((_MLP_ROWS,), lambda i: (i,)),
        out_shape=jax.ShapeDtypeStruct((n,), jnp.float32),
    )(x, W1, b1.reshape(H, 1), W2, b2.reshape(1, 1))


def _sc_mesh():
    return plsc.VectorSubcoreMesh(
        core_axis_name="c", subcore_axis_name="s",
        num_cores=_NC, num_subcores=_NS)


def _sc_accumulate(vmov, msrc2, mdst2, pi2, adv2):
    out_type = (
        jax.ShapeDtypeStruct((_NC * _SEG_PAD,), jnp.float32),
        jax.ShapeDtypeStruct((_NC * _SEG_PAD,), jnp.float32),
        jax.ShapeDtypeStruct((_NC * _SEG_PAD,), jnp.float32),
    )

    @functools.partial(
        pl.kernel, out_type=out_type, mesh=_sc_mesh(),
        scratch_types=[
            pltpu.VMEM((_ECH,), jnp.int32),
            pltpu.VMEM((_ECH,), jnp.int32),
            pltpu.VMEM((_ECH,), jnp.float32),
            pltpu.VMEM((_PCH,), jnp.int32),
            pltpu.VMEM((_PCH,), jnp.float32),
            pltpu.VMEM((_SEG_TILE,), jnp.float32),
            pltpu.VMEM_SHARED((_SEG_PAD,), jnp.float32),
            pltpu.VMEM_SHARED((_SEG_PAD,), jnp.float32),
            pltpu.VMEM_SHARED((_SEG_PAD,), jnp.float32),
            pltpu.SemaphoreType.DMA,
        ])
    def k(vmov_h, msrc_h, mdst_h, pi_h, adv_h, svp_h, psp_h, pcp_h,
          bidx, bdst, bval, bpdst, bpval, bzero, acc_sv, acc_ps, acc_pc, sem):
        c = lax.axis_index("c")
        s = lax.axis_index("s")
        w = s * _NC + c

        def zfill(i, carry):
            bzero[pl.ds(i * 16, 16)] = jnp.zeros((16,), jnp.float32)
            return carry
        lax.fori_loop(0, _SEG_TILE // 16, zfill, 0)

        tile_sl = pl.ds(s * _SEG_TILE, _SEG_TILE)
        pltpu.sync_copy(bzero, acc_sv.at[tile_sl])
        pltpu.sync_copy(bzero, acc_ps.at[tile_sl])
        pltpu.sync_copy(bzero, acc_pc.at[tile_sl])
        plsc.subcore_barrier()

        pltpu.sync_copy(msrc_h.at[pl.ds(w * _ECH, _ECH)], bidx)
        pltpu.sync_copy(mdst_h.at[pl.ds(w * _ECH, _ECH)], bdst)
        pltpu.async_copy(vmov_h.at[bidx], bval, sem).wait()
        pltpu.sync_copy(bval, acc_sv.at[bdst], add=True)

        pltpu.sync_copy(pi_h.at[pl.ds(w * _PCH, _PCH)], bpdst)
        pltpu.sync_copy(adv_h.at[pl.ds(w * _PCH, _PCH)], bpval)
        pltpu.sync_copy(bpval, acc_ps.at[bpdst], add=True)

        def ofill(i, carry):
            bpval[pl.ds(i * 16, 16)] = jnp.ones((16,), jnp.float32)
            return carry
        lax.fori_loop(0, _PCH // 16, ofill, 0)
        pltpu.sync_copy(bpval, acc_pc.at[bpdst], add=True)

        plsc.subcore_barrier()
        out_sl = pl.ds(c * _SEG_PAD + s * _SEG_TILE, _SEG_TILE)
        pltpu.sync_copy(acc_sv.at[tile_sl], bzero)
        pltpu.sync_copy(bzero, svp_h.at[out_sl])
        pltpu.sync_copy(acc_ps.at[tile_sl], bzero)
        pltpu.sync_copy(bzero, psp_h.at[out_sl])
        pltpu.sync_copy(acc_pc.at[tile_sl], bzero)
        pltpu.sync_copy(bzero, pcp_h.at[out_sl])

    return k(vmov, msrc2, mdst2, pi2, adv2)


def _sc_combine(svp, psp, pcp):
    @functools.partial(
        pl.kernel, out_type=jax.ShapeDtypeStruct((_SEG_PAD,), jnp.float32),
        mesh=_sc_mesh(),
        scratch_types=[pltpu.VMEM((_SEG_W,), jnp.float32) for _ in range(7)])
    def k(svp_h, psp_h, pcp_h, t_h, a0, a1, p0, p1, c0, c1, tb):
        c = lax.axis_index("c")
        s = lax.axis_index("s")
        w = s * _NC + c
        base = w * _SEG_W
        pltpu.sync_copy(svp_h.at[pl.ds(base, _SEG_W)], a0)
        pltpu.sync_copy(svp_h.at[pl.ds(_SEG_PAD + base, _SEG_W)], a1)
        pltpu.sync_copy(psp_h.at[pl.ds(base, _SEG_W)], p0)
        pltpu.sync_copy(psp_h.at[pl.ds(_SEG_PAD + base, _SEG_W)], p1)
        pltpu.sync_copy(pcp_h.at[pl.ds(base, _SEG_W)], c0)
        pltpu.sync_copy(pcp_h.at[pl.ds(_SEG_PAD + base, _SEG_W)], c1)

        def step(i, carry):
            sl = pl.ds(i * 16, 16)
            sv = a0[sl] + a1[sl]
            ps = p0[sl] + p1[sl]
            pc = c0[sl] + c1[sl]
            tb[sl] = sv - ps / jnp.maximum(pc, 1.0)
            return carry
        lax.fori_loop(0, _SEG_W // 16, step, 0)
        pltpu.sync_copy(tb, t_h.at[pl.ds(base, _SEG_W)])

    return k(svp, psp, pcp)


def _sc_gather_out(t, pi1, adv1):
    @functools.partial(
        pl.kernel, out_type=jax.ShapeDtypeStruct((_P_PAD,), jnp.float32),
        mesh=_sc_mesh(),
        scratch_types=[
            pltpu.VMEM((_PCH,), jnp.int32),
            pltpu.VMEM((_PCH,), jnp.float32),
            pltpu.VMEM((_PCH,), jnp.float32),
            pltpu.SemaphoreType.DMA,
        ])
    def k(t_h, pi_h, adv_h, out_h, bpi, badv, btv, sem):
        c = lax.axis_index("c")
        s = lax.axis_index("s")
        w = s * _NC + c
        sl = pl.ds(w * _PCH, _PCH)
        pltpu.sync_copy(pi_h.at[sl], bpi)
        pltpu.sync_copy(adv_h.at[sl], badv)
        pltpu.async_copy(t_h.at[bpi], btv, sem).wait()

        def astep(i, carry):
            ssl = pl.ds(i * 16, 16)
            badv[ssl] = badv[ssl] + btv[ssl]
            return carry
        lax.fori_loop(0, _PCH // 16, astep, 0)
        pltpu.sync_copy(badv, out_h.at[sl])

    return k(t, pi1, adv1)


def kernel(x_movement, x_phase, mi_src, mi_dst, pi_dst, W1, b1, W2, b2):
    v_mov = _mlp(x_movement, W1, b1, W2, b2)
    adv = _mlp(x_phase, W1, b1, W2, b2)

    e_pad = _E_PAD - N_MOV
    p_pad = _P_PAD - N_PHASE
    msrc1 = jnp.concatenate([mi_src, jnp.zeros((e_pad,), jnp.int32)])
    mdst1 = jnp.concatenate([mi_dst, jnp.full((e_pad,), _DUMMY_SEG, jnp.int32)])
    pi1 = jnp.concatenate([pi_dst, jnp.full((p_pad,), _DUMMY_SEG, jnp.int32)])
    adv1 = jnp.concatenate([adv, jnp.zeros((p_pad,), jnp.float32)])

    svp, psp, pcp = _sc_accumulate(v_mov, msrc1, mdst1, pi1, adv1)
    t = _sc_combine(svp, psp, pcp)
    out_p = _sc_gather_out(t, pi1, adv1)
    return out_p[:N_PHASE], pi_dst


# speedup vs baseline: 40.9406x; 1.6071x over previous
import functools

import jax
import jax.numpy as jnp
from jax import lax
from jax.experimental import pallas as pl
from jax.experimental.pallas import tpu as pltpu
from jax.experimental.pallas import tpu_sc as plsc

N_MOV = 600000
N_PHASE = 400000
N_INT = 50000
D = 128
H = 256

_NC = 2
_NS = 16
_NW = _NC * _NS

_EW = 152
_PW = 104
_E_PAD = _NW * _EW * 128
_P_PAD = _NW * _PW * 128
_SEG_PAD = 50176
_SEG_TILE = _SEG_PAD // _NS
_SEG_W = _SEG_PAD // _NW
_DUMMY_SEG = 50100
_ECH = _EW * 128
_PCH = _PW * 128
_V_PAD = 600064
_VCH = _V_PAD // 32

_MLP_ROWS = 4096


def _mlp_block(x_ref, w1_ref, b1_ref, w2_ref, b2_ref, o_ref):
    x = x_ref[...].astype(jnp.bfloat16)
    w1 = w1_ref[...].astype(jnp.bfloat16)
    hT = lax.dot_general(w1, x, dimension_numbers=(((0,), (1,)), ((), ())),
                         preferred_element_type=jnp.float32)
    hT = jnp.maximum(hT + b1_ref[...], 0.0)
    o_ref[...] = jnp.sum(hT * w2_ref[...], axis=0) + b2_ref[0, 0]


def _mlp(x, W1, b1, W2, b2):
    n = x.shape[0]
    return pl.pallas_call(
        _mlp_block,
        grid=((n + _MLP_ROWS - 1) // _MLP_ROWS,),
        in_specs=[
            pl.BlockSpec((_MLP_ROWS, D), lambda i: (i, 0)),
            pl.BlockSpec((D, H), lambda i: (0, 0)),
            pl.BlockSpec((H, 1), lambda i: (0, 0)),
            pl.BlockSpec((H, 1), lambda i: (0, 0)),
            pl.BlockSpec((1, 1), lambda i: (0, 0)),
        ],
        out_specs=pl.BlockSpec((_MLP_ROWS,), lambda i: (i,)),
        out_shape=jax.ShapeDtypeStruct((n,), jnp.float32),
    )(x, W1, b1.reshape(H, 1), W2, b2.reshape(1, 1))


def _sc_mesh():
    return plsc.VectorSubcoreMesh(
        core_axis_name="c", subcore_axis_name="s",
        num_cores=_NC, num_subcores=_NS)


def _sc_phase_accumulate(pi1, adv1):
    out_type = (
        jax.ShapeDtypeStruct((_NC * _SEG_PAD,), jnp.float32),
        jax.ShapeDtypeStruct((_NC * _SEG_PAD,), jnp.float32),
    )

    @functools.partial(
        pl.kernel, out_type=out_type, mesh=_sc_mesh(),
        scratch_types=[
            pltpu.VMEM((_PCH,), jnp.int32),
            pltpu.VMEM((_PCH,), jnp.float32),
            pltpu.VMEM((_SEG_TILE,), jnp.float32),
            pltpu.VMEM_SHARED((_SEG_PAD,), jnp.float32),
            pltpu.VMEM_SHARED((_SEG_PAD,), jnp.float32),
        ])
    def k(pi_h, adv_h, psp_h, pcp_h, bpdst, bpval, bzero, acc_ps, acc_pc):
        c = lax.axis_index("c")
        s = lax.axis_index("s")
        w = s * _NC + c

        def zfill(i, carry):
            bzero[pl.ds(i * 16, 16)] = jnp.zeros((16,), jnp.float32)
            return carry
        lax.fori_loop(0, _SEG_TILE // 16, zfill, 0)

        tile_sl = pl.ds(s * _SEG_TILE, _SEG_TILE)
        pltpu.sync_copy(bzero, acc_ps.at[tile_sl])
        pltpu.sync_copy(bzero, acc_pc.at[tile_sl])
        pltpu.sync_copy(pi_h.at[pl.ds(w * _PCH, _PCH)], bpdst)
        pltpu.sync_copy(adv_h.at[pl.ds(w * _PCH, _PCH)], bpval)
        plsc.subcore_barrier()

        pltpu.sync_copy(bpval, acc_ps.at[bpdst], add=True)

        def ofill(i, carry):
            bpval[pl.ds(i * 16, 16)] = jnp.ones((16,), jnp.float32)
            return carry
        lax.fori_loop(0, _PCH // 16, ofill, 0)
        pltpu.sync_copy(bpval, acc_pc.at[bpdst], add=True)

        plsc.subcore_barrier()
        out_sl = pl.ds(c * _SEG_PAD + s * _SEG_TILE, _SEG_TILE)
        pltpu.sync_copy(acc_ps.at[tile_sl], bzero)
        pltpu.sync_copy(bzero, psp_h.at[out_sl])
        pltpu.sync_copy(acc_pc.at[tile_sl], bzero)
        pltpu.sync_copy(bzero, pcp_h.at[out_sl])

    return k(pi1, adv1)


def _sc_edge_accumulate(vmov, msrc1, mdst1):
    @functools.partial(
        pl.kernel,
        out_type=jax.ShapeDtypeStruct((_NC * _SEG_PAD,), jnp.float32),
        mesh=_sc_mesh(),
        scratch_types=[
            pltpu.VMEM((_ECH,), jnp.int32),
            pltpu.VMEM((_ECH,), jnp.int32),
            pltpu.VMEM((_ECH,), jnp.float32),
            pltpu.VMEM((_SEG_TILE,), jnp.float32),
            pltpu.VMEM_SHARED((_SEG_PAD,), jnp.float32),
            pltpu.VMEM_SHARED((_V_PAD,), jnp.float32),
            pltpu.SemaphoreType.DMA,
        ])
    def k(vmov_h, msrc_h, mdst_h, svp_h, bidx, bdst, bval, bzero,
          acc_sv, vms, sem):
        c = lax.axis_index("c")
        s = lax.axis_index("s")
        w = s * _NC + c

        def zfill(i, carry):
            bzero[pl.ds(i * 16, 16)] = jnp.zeros((16,), jnp.float32)
            return carry
        lax.fori_loop(0, _SEG_TILE // 16, zfill, 0)

        tile_sl = pl.ds(s * _SEG_TILE, _SEG_TILE)
        pltpu.sync_copy(bzero, acc_sv.at[tile_sl])

        for kk in range(2):
            st_sl = pl.ds(s * (2 * _VCH) + kk * _VCH, _VCH)
            pltpu.sync_copy(vmov_h.at[st_sl], bval.at[pl.ds(0, _VCH)])
            pltpu.sync_copy(bval.at[pl.ds(0, _VCH)], vms.at[st_sl])

        pltpu.sync_copy(msrc_h.at[pl.ds(w * _ECH, _ECH)], bidx)
        pltpu.sync_copy(mdst_h.at[pl.ds(w * _ECH, _ECH)], bdst)
        plsc.subcore_barrier()

        pltpu.async_copy(vms.at[bidx], bval, sem).wait()
        pltpu.sync_copy(bval, acc_sv.at[bdst], add=True)

        plsc.subcore_barrier()
        out_sl = pl.ds(c * _SEG_PAD + s * _SEG_TILE, _SEG_TILE)
        pltpu.sync_copy(acc_sv.at[tile_sl], bzero)
        pltpu.sync_copy(bzero, svp_h.at[out_sl])

    return k(vmov, msrc1, mdst1)


def _sc_combine(svp, psp, pcp):
    @functools.partial(
        pl.kernel, out_type=jax.ShapeDtypeStruct((_SEG_PAD,), jnp.float32),
        mesh=_sc_mesh(),
        scratch_types=[pltpu.VMEM((_SEG_W,), jnp.float32) for _ in range(7)])
    def k(svp_h, psp_h, pcp_h, t_h, a0, a1, p0, p1, c0, c1, tb):
        c = lax.axis_index("c")
        s = lax.axis_index("s")
        w = s * _NC + c
        base = w * _SEG_W
        pltpu.sync_copy(svp_h.at[pl.ds(base, _SEG_W)], a0)
        pltpu.sync_copy(svp_h.at[pl.ds(_SEG_PAD + base, _SEG_W)], a1)
        pltpu.sync_copy(psp_h.at[pl.ds(base, _SEG_W)], p0)
        pltpu.sync_copy(psp_h.at[pl.ds(_SEG_PAD + base, _SEG_W)], p1)
        pltpu.sync_copy(pcp_h.at[pl.ds(base, _SEG_W)], c0)
        pltpu.sync_copy(pcp_h.at[pl.ds(_SEG_PAD + base, _SEG_W)], c1)

        def step(i, carry):
            sl = pl.ds(i * 16, 16)
            sv = a0[sl] + a1[sl]
            ps = p0[sl] + p1[sl]
            pc = c0[sl] + c1[sl]
            tb[sl] = sv - ps / jnp.maximum(pc, 1.0)
            return carry
        lax.fori_loop(0, _SEG_W // 16, step, 0)
        pltpu.sync_copy(tb, t_h.at[pl.ds(base, _SEG_W)])

    return k(svp, psp, pcp)


def _sc_gather_out(t, pi1, adv1):
    @functools.partial(
        pl.kernel, out_type=jax.ShapeDtypeStruct((_P_PAD,), jnp.float32),
        mesh=_sc_mesh(),
        scratch_types=[
            pltpu.VMEM((_PCH,), jnp.int32),
            pltpu.VMEM((_PCH,), jnp.float32),
            pltpu.VMEM((_PCH,), jnp.float32),
            pltpu.VMEM((_SEG_TILE,), jnp.float32),
            pltpu.VMEM_SHARED((_SEG_PAD,), jnp.float32),
            pltpu.SemaphoreType.DMA,
        ])
    def k(t_h, pi_h, adv_h, out_h, bpi, badv, btv, bstage, bts, sem):
        c = lax.axis_index("c")
        s = lax.axis_index("s")
        w = s * _NC + c
        sl = pl.ds(w * _PCH, _PCH)
        st_sl = pl.ds(s * _SEG_TILE, _SEG_TILE)
        pltpu.sync_copy(t_h.at[st_sl], bstage)
        pltpu.sync_copy(bstage, bts.at[st_sl])
        pltpu.sync_copy(pi_h.at[sl], bpi)
        pltpu.sync_copy(adv_h.at[sl], badv)
        plsc.subcore_barrier()
        pltpu.async_copy(bts.at[bpi], btv, sem).wait()

        def astep(i, carry):
            ssl = pl.ds(i * 16, 16)
            badv[ssl] = badv[ssl] + btv[ssl]
            return carry
        lax.fori_loop(0, _PCH // 16, astep, 0)
        pltpu.sync_copy(badv, out_h.at[sl])

    return k(t, pi1, adv1)


def kernel(x_movement, x_phase, mi_src, mi_dst, pi_dst, W1, b1, W2, b2):
    adv = _mlp(x_phase, W1, b1, W2, b2)
    p_pad = _P_PAD - N_PHASE
    pi1 = jnp.concatenate([pi_dst, jnp.full((p_pad,), _DUMMY_SEG, jnp.int32)])
    adv1 = jnp.concatenate([adv, jnp.zeros((p_pad,), jnp.float32)])
    psp, pcp = _sc_phase_accumulate(pi1, adv1)

    v_mov = _mlp(x_movement, W1, b1, W2, b2)
    v_mov = jnp.concatenate([v_mov, jnp.zeros((_V_PAD - N_MOV,), jnp.float32)])
    e_pad = _E_PAD - N_MOV
    msrc1 = jnp.concatenate([mi_src, jnp.zeros((e_pad,), jnp.int32)])
    mdst1 = jnp.concatenate([mi_dst, jnp.full((e_pad,), _DUMMY_SEG, jnp.int32)])
    svp = _sc_edge_accumulate(v_mov, msrc1, mdst1)

    t = _sc_combine(svp, psp, pcp)
    out_p = _sc_gather_out(t, pi1, adv1)
    return out_p[:N_PHASE], pi_dst


# speedup vs baseline: 49.7456x; 1.2151x over previous
import functools

import jax
import jax.numpy as jnp
from jax import lax
from jax.experimental import pallas as pl
from jax.experimental.pallas import tpu as pltpu
from jax.experimental.pallas import tpu_sc as plsc

N_MOV = 600000
N_PHASE = 400000
N_INT = 50000
D = 128
H = 256

_NC = 2
_NS = 16
_NW = _NC * _NS

_EW = 152
_PW = 104
_E_PAD = _NW * _EW * 128
_P_PAD = _NW * _PW * 128
_SEG_PAD = 50176
_SEG_TILE = _SEG_PAD // _NS
_SEG_W = _SEG_PAD // _NW
_DUMMY_SEG = 50100
_ECH = _EW * 128
_PCH = _PW * 128
_V_PAD = 600064
_VCH = _V_PAD // 32

_MLP_ROWS = 8192


def _mlp_block(x_ref, w1_ref, b1_ref, w2_ref, b2_ref, o_ref):
    x = x_ref[...].astype(jnp.bfloat16)
    w1 = w1_ref[...].astype(jnp.bfloat16)
    hT = lax.dot_general(w1, x, dimension_numbers=(((0,), (1,)), ((), ())),
                         preferred_element_type=jnp.float32)
    hT = jnp.maximum(hT + b1_ref[...], 0.0)
    o_ref[...] = jnp.sum(hT * w2_ref[...], axis=0) + b2_ref[0, 0]


def _mlp(x, W1, b1, W2, b2):
    n = x.shape[0]
    return pl.pallas_call(
        _mlp_block,
        grid=((n + _MLP_ROWS - 1) // _MLP_ROWS,),
        in_specs=[
            pl.BlockSpec((_MLP_ROWS, D), lambda i: (i, 0)),
            pl.BlockSpec((D, H), lambda i: (0, 0)),
            pl.BlockSpec((H, 1), lambda i: (0, 0)),
            pl.BlockSpec((H, 1), lambda i: (0, 0)),
            pl.BlockSpec((1, 1), lambda i: (0, 0)),
        ],
        out_specs=pl.BlockSpec((_MLP_ROWS,), lambda i: (i,)),
        out_shape=jax.ShapeDtypeStruct((n,), jnp.float32),
    )(x, W1, b1.reshape(H, 1), W2, b2.reshape(1, 1))


def _sc_mesh():
    return plsc.VectorSubcoreMesh(
        core_axis_name="c", subcore_axis_name="s",
        num_cores=_NC, num_subcores=_NS)


def _sc_phase_accumulate(pi1, adv1):
    out_type = (
        jax.ShapeDtypeStruct((_NC * _SEG_PAD,), jnp.float32),
        jax.ShapeDtypeStruct((_NC * _SEG_PAD,), jnp.float32),
    )

    @functools.partial(
        pl.kernel, out_type=out_type, mesh=_sc_mesh(),
        scratch_types=[
            pltpu.VMEM((_PCH,), jnp.int32),
            pltpu.VMEM((_PCH,), jnp.float32),
            pltpu.VMEM((_SEG_TILE,), jnp.float32),
            pltpu.VMEM_SHARED((_SEG_PAD,), jnp.float32),
            pltpu.VMEM_SHARED((_SEG_PAD,), jnp.float32),
        ])
    def k(pi_h, adv_h, psp_h, pcp_h, bpdst, bpval, bzero, acc_ps, acc_pc):
        c = lax.axis_index("c")
        s = lax.axis_index("s")
        w = s * _NC + c

        def zfill(i, carry):
            bzero[pl.ds(i * 16, 16)] = jnp.zeros((16,), jnp.float32)
            return carry
        lax.fori_loop(0, _SEG_TILE // 16, zfill, 0)

        tile_sl = pl.ds(s * _SEG_TILE, _SEG_TILE)
        pltpu.sync_copy(bzero, acc_ps.at[tile_sl])
        pltpu.sync_copy(bzero, acc_pc.at[tile_sl])
        pltpu.sync_copy(pi_h.at[pl.ds(w * _PCH, _PCH)], bpdst)
        pltpu.sync_copy(adv_h.at[pl.ds(w * _PCH, _PCH)], bpval)
        plsc.subcore_barrier()

        pltpu.sync_copy(bpval, acc_ps.at[bpdst], add=True)

        def ofill(i, carry):
            bpval[pl.ds(i * 16, 16)] = jnp.ones((16,), jnp.float32)
            return carry
        lax.fori_loop(0, _PCH // 16, ofill, 0)
        pltpu.sync_copy(bpval, acc_pc.at[bpdst], add=True)

        plsc.subcore_barrier()
        out_sl = pl.ds(c * _SEG_PAD + s * _SEG_TILE, _SEG_TILE)
        pltpu.sync_copy(acc_ps.at[tile_sl], bzero)
        pltpu.sync_copy(bzero, psp_h.at[out_sl])
        pltpu.sync_copy(acc_pc.at[tile_sl], bzero)
        pltpu.sync_copy(bzero, pcp_h.at[out_sl])

    return k(pi1, adv1)


def _sc_edge_accumulate(vmov, msrc1, mdst1):
    @functools.partial(
        pl.kernel,
        out_type=jax.ShapeDtypeStruct((_NC * _SEG_PAD,), jnp.float32),
        mesh=_sc_mesh(),
        scratch_types=[
            pltpu.VMEM((_ECH,), jnp.int32),
            pltpu.VMEM((_ECH,), jnp.int32),
            pltpu.VMEM((_ECH,), jnp.float32),
            pltpu.VMEM((_SEG_TILE,), jnp.float32),
            pltpu.VMEM_SHARED((_SEG_PAD,), jnp.float32),
            pltpu.VMEM_SHARED((_V_PAD,), jnp.float32),
            pltpu.SemaphoreType.DMA,
        ])
    def k(vmov_h, msrc_h, mdst_h, svp_h, bidx, bdst, bval, bzero,
          acc_sv, vms, sem):
        c = lax.axis_index("c")
        s = lax.axis_index("s")
        w = s * _NC + c

        def zfill(i, carry):
            bzero[pl.ds(i * 16, 16)] = jnp.zeros((16,), jnp.float32)
            return carry
        lax.fori_loop(0, _SEG_TILE // 16, zfill, 0)

        tile_sl = pl.ds(s * _SEG_TILE, _SEG_TILE)
        pltpu.sync_copy(bzero, acc_sv.at[tile_sl])

        for kk in range(2):
            st_sl = pl.ds(s * (2 * _VCH) + kk * _VCH, _VCH)
            pltpu.sync_copy(vmov_h.at[st_sl], bval.at[pl.ds(0, _VCH)])
            pltpu.sync_copy(bval.at[pl.ds(0, _VCH)], vms.at[st_sl])

        pltpu.sync_copy(msrc_h.at[pl.ds(w * _ECH, _ECH)], bidx)
        pltpu.sync_copy(mdst_h.at[pl.ds(w * _ECH, _ECH)], bdst)
        plsc.subcore_barrier()

        pltpu.async_copy(vms.at[bidx], bval, sem).wait()
        pltpu.sync_copy(bval, acc_sv.at[bdst], add=True)

        plsc.subcore_barrier()
        out_sl = pl.ds(c * _SEG_PAD + s * _SEG_TILE, _SEG_TILE)
        pltpu.sync_copy(acc_sv.at[tile_sl], bzero)
        pltpu.sync_copy(bzero, svp_h.at[out_sl])

    return k(vmov, msrc1, mdst1)


def _sc_combine(svp, psp, pcp):
    @functools.partial(
        pl.kernel, out_type=jax.ShapeDtypeStruct((_SEG_PAD,), jnp.float32),
        mesh=_sc_mesh(),
        scratch_types=[pltpu.VMEM((_SEG_W,), jnp.float32) for _ in range(7)])
    def k(svp_h, psp_h, pcp_h, t_h, a0, a1, p0, p1, c0, c1, tb):
        c = lax.axis_index("c")
        s = lax.axis_index("s")
        w = s * _NC + c
        base = w * _SEG_W
        pltpu.sync_copy(svp_h.at[pl.ds(base, _SEG_W)], a0)
        pltpu.sync_copy(svp_h.at[pl.ds(_SEG_PAD + base, _SEG_W)], a1)
        pltpu.sync_copy(psp_h.at[pl.ds(base, _SEG_W)], p0)
        pltpu.sync_copy(psp_h.at[pl.ds(_SEG_PAD + base, _SEG_W)], p1)
        pltpu.sync_copy(pcp_h.at[pl.ds(base, _SEG_W)], c0)
        pltpu.sync_copy(pcp_h.at[pl.ds(_SEG_PAD + base, _SEG_W)], c1)

        def step(i, carry):
            sl = pl.ds(i * 16, 16)
            sv = a0[sl] + a1[sl]
            ps = p0[sl] + p1[sl]
            pc = c0[sl] + c1[sl]
            tb[sl] = sv - ps / jnp.maximum(pc, 1.0)
            return carry
        lax.fori_loop(0, _SEG_W // 16, step, 0)
        pltpu.sync_copy(tb, t_h.at[pl.ds(base, _SEG_W)])

    return k(svp, psp, pcp)


def _sc_gather_out(t, pi1, adv1):
    @functools.partial(
        pl.kernel, out_type=jax.ShapeDtypeStruct((_P_PAD,), jnp.float32),
        mesh=_sc_mesh(),
        scratch_types=[
            pltpu.VMEM((_PCH,), jnp.int32),
            pltpu.VMEM((_PCH,), jnp.float32),
            pltpu.VMEM((_PCH,), jnp.float32),
            pltpu.VMEM((_SEG_TILE,), jnp.float32),
            pltpu.VMEM_SHARED((_SEG_PAD,), jnp.float32),
            pltpu.SemaphoreType.DMA,
        ])
    def k(t_h, pi_h, adv_h, out_h, bpi, badv, btv, bstage, bts, sem):
        c = lax.axis_index("c")
        s = lax.axis_index("s")
        w = s * _NC + c
        sl = pl.ds(w * _PCH, _PCH)
        st_sl = pl.ds(s * _SEG_TILE, _SEG_TILE)
        pltpu.sync_copy(t_h.at[st_sl], bstage)
        pltpu.sync_copy(bstage, bts.at[st_sl])
        pltpu.sync_copy(pi_h.at[sl], bpi)
        pltpu.sync_copy(adv_h.at[sl], badv)
        plsc.subcore_barrier()
        pltpu.async_copy(bts.at[bpi], btv, sem).wait()

        def astep(i, carry):
            ssl = pl.ds(i * 16, 16)
            badv[ssl] = badv[ssl] + btv[ssl]
            return carry
        lax.fori_loop(0, _PCH // 16, astep, 0)
        pltpu.sync_copy(badv, out_h.at[sl])

    return k(t, pi1, adv1)


def kernel(x_movement, x_phase, mi_src, mi_dst, pi_dst, W1, b1, W2, b2):
    adv = _mlp(x_phase, W1, b1, W2, b2)
    p_pad = _P_PAD - N_PHASE
    pi1 = jnp.concatenate([pi_dst, jnp.full((p_pad,), _DUMMY_SEG, jnp.int32)])
    adv1 = jnp.concatenate([adv, jnp.zeros((p_pad,), jnp.float32)])
    psp, pcp = _sc_phase_accumulate(pi1, adv1)

    v_mov = _mlp(x_movement, W1, b1, W2, b2)
    v_mov = jnp.concatenate([v_mov, jnp.zeros((_V_PAD - N_MOV,), jnp.float32)])
    e_pad = _E_PAD - N_MOV
    msrc1 = jnp.concatenate([mi_src, jnp.zeros((e_pad,), jnp.int32)])
    mdst1 = jnp.concatenate([mi_dst, jnp.full((e_pad,), _DUMMY_SEG, jnp.int32)])
    svp = _sc_edge_accumulate(v_mov, msrc1, mdst1)

    t = _sc_combine(svp, psp, pcp)
    out_p = _sc_gather_out(t, pi1, adv1)
    return out_p[:N_PHASE], pi_dst


# speedup vs baseline: 55.9326x; 1.1244x over previous
import functools

import jax
import jax.numpy as jnp
from jax import lax
from jax.experimental import pallas as pl
from jax.experimental.pallas import tpu as pltpu
from jax.experimental.pallas import tpu_sc as plsc

N_MOV = 600000
N_PHASE = 400000
N_INT = 50000
D = 128
H = 256

_NC = 2
_NS = 16
_NW = _NC * _NS

_EW = 152
_PW = 104
_E_PAD = _NW * _EW * 128
_P_PAD = _NW * _PW * 128
_SEG_PAD = 50176
_SEG_TILE = _SEG_PAD // _NS
_SEG_W = _SEG_PAD // _NW
_DUMMY_SEG = 50100
_ECH = _EW * 128
_PCH = _PW * 128
_V_PAD = 600064
_VCH = _V_PAD // 32

_MLP_ROWS = 16384


def _mlp_block(x_ref, w1_ref, b1_ref, w2_ref, b2_ref, o_ref):
    x = x_ref[...].astype(jnp.bfloat16)
    w1 = w1_ref[...].astype(jnp.bfloat16)
    hT = lax.dot_general(w1, x, dimension_numbers=(((0,), (1,)), ((), ())),
                         preferred_element_type=jnp.float32)
    hT = jnp.maximum(hT + b1_ref[...], 0.0)
    o_ref[...] = jnp.sum(hT * w2_ref[...], axis=0) + b2_ref[0, 0]


def _mlp(x, W1, b1, W2, b2):
    n = x.shape[0]
    return pl.pallas_call(
        _mlp_block,
        grid=((n + _MLP_ROWS - 1) // _MLP_ROWS,),
        in_specs=[
            pl.BlockSpec((_MLP_ROWS, D), lambda i: (i, 0)),
            pl.BlockSpec((D, H), lambda i: (0, 0)),
            pl.BlockSpec((H, 1), lambda i: (0, 0)),
            pl.BlockSpec((H, 1), lambda i: (0, 0)),
            pl.BlockSpec((1, 1), lambda i: (0, 0)),
        ],
        out_specs=pl.BlockSpec((_MLP_ROWS,), lambda i: (i,)),
        out_shape=jax.ShapeDtypeStruct((n,), jnp.float32),
    )(x, W1, b1.reshape(H, 1), W2, b2.reshape(1, 1))


def _sc_mesh():
    return plsc.VectorSubcoreMesh(
        core_axis_name="c", subcore_axis_name="s",
        num_cores=_NC, num_subcores=_NS)


def _sc_phase_accumulate(pi1, adv1):
    out_type = (
        jax.ShapeDtypeStruct((_NC * _SEG_PAD,), jnp.float32),
        jax.ShapeDtypeStruct((_NC * _SEG_PAD,), jnp.float32),
    )

    @functools.partial(
        pl.kernel, out_type=out_type, mesh=_sc_mesh(),
        scratch_types=[
            pltpu.VMEM((_PCH,), jnp.int32),
            pltpu.VMEM((_PCH,), jnp.float32),
            pltpu.VMEM((_SEG_TILE,), jnp.float32),
            pltpu.VMEM_SHARED((_SEG_PAD,), jnp.float32),
            pltpu.VMEM_SHARED((_SEG_PAD,), jnp.float32),
        ])
    def k(pi_h, adv_h, psp_h, pcp_h, bpdst, bpval, bzero, acc_ps, acc_pc):
        c = lax.axis_index("c")
        s = lax.axis_index("s")
        w = s * _NC + c

        def zfill(i, carry):
            bzero[pl.ds(i * 16, 16)] = jnp.zeros((16,), jnp.float32)
            return carry
        lax.fori_loop(0, _SEG_TILE // 16, zfill, 0)

        tile_sl = pl.ds(s * _SEG_TILE, _SEG_TILE)
        pltpu.sync_copy(bzero, acc_ps.at[tile_sl])
        pltpu.sync_copy(bzero, acc_pc.at[tile_sl])
        pltpu.sync_copy(pi_h.at[pl.ds(w * _PCH, _PCH)], bpdst)
        pltpu.sync_copy(adv_h.at[pl.ds(w * _PCH, _PCH)], bpval)
        plsc.subcore_barrier()

        pltpu.sync_copy(bpval, acc_ps.at[bpdst], add=True)

        def ofill(i, carry):
            bpval[pl.ds(i * 16, 16)] = jnp.ones((16,), jnp.float32)
            return carry
        lax.fori_loop(0, _PCH // 16, ofill, 0)
        pltpu.sync_copy(bpval, acc_pc.at[bpdst], add=True)

        plsc.subcore_barrier()
        out_sl = pl.ds(c * _SEG_PAD + s * _SEG_TILE, _SEG_TILE)
        pltpu.sync_copy(acc_ps.at[tile_sl], bzero)
        pltpu.sync_copy(bzero, psp_h.at[out_sl])
        pltpu.sync_copy(acc_pc.at[tile_sl], bzero)
        pltpu.sync_copy(bzero, pcp_h.at[out_sl])

    return k(pi1, adv1)


def _sc_edge_accumulate(vmov, msrc1, mdst1):
    @functools.partial(
        pl.kernel,
        out_type=jax.ShapeDtypeStruct((_NC * _SEG_PAD,), jnp.float32),
        mesh=_sc_mesh(),
        scratch_types=[
            pltpu.VMEM((_ECH,), jnp.int32),
            pltpu.VMEM((_ECH,), jnp.int32),
            pltpu.VMEM((_ECH,), jnp.float32),
            pltpu.VMEM((_SEG_TILE,), jnp.float32),
            pltpu.VMEM_SHARED((_SEG_PAD,), jnp.float32),
            pltpu.VMEM_SHARED((_V_PAD,), jnp.float32),
            pltpu.SemaphoreType.DMA,
        ])
    def k(vmov_h, msrc_h, mdst_h, svp_h, bidx, bdst, bval, bzero,
          acc_sv, vms, sem):
        c = lax.axis_index("c")
        s = lax.axis_index("s")
        w = s * _NC + c

        def zfill(i, carry):
            bzero[pl.ds(i * 16, 16)] = jnp.zeros((16,), jnp.float32)
            return carry
        lax.fori_loop(0, _SEG_TILE // 16, zfill, 0)

        tile_sl = pl.ds(s * _SEG_TILE, _SEG_TILE)
        pltpu.sync_copy(bzero, acc_sv.at[tile_sl])

        for kk in range(2):
            st_sl = pl.ds(s * (2 * _VCH) + kk * _VCH, _VCH)
            pltpu.sync_copy(vmov_h.at[st_sl], bval.at[pl.ds(0, _VCH)])
            pltpu.sync_copy(bval.at[pl.ds(0, _VCH)], vms.at[st_sl])

        pltpu.sync_copy(msrc_h.at[pl.ds(w * _ECH, _ECH)], bidx)
        pltpu.sync_copy(mdst_h.at[pl.ds(w * _ECH, _ECH)], bdst)
        plsc.subcore_barrier()

        pltpu.async_copy(vms.at[bidx], bval, sem).wait()
        pltpu.sync_copy(bval, acc_sv.at[bdst], add=True)

        plsc.subcore_barrier()
        out_sl = pl.ds(c * _SEG_PAD + s * _SEG_TILE, _SEG_TILE)
        pltpu.sync_copy(acc_sv.at[tile_sl], bzero)
        pltpu.sync_copy(bzero, svp_h.at[out_sl])

    return k(vmov, msrc1, mdst1)


def _sc_combine(svp, psp, pcp):
    @functools.partial(
        pl.kernel, out_type=jax.ShapeDtypeStruct((_SEG_PAD,), jnp.float32),
        mesh=_sc_mesh(),
        scratch_types=[pltpu.VMEM((_SEG_W,), jnp.float32) for _ in range(7)])
    def k(svp_h, psp_h, pcp_h, t_h, a0, a1, p0, p1, c0, c1, tb):
        c = lax.axis_index("c")
        s = lax.axis_index("s")
        w = s * _NC + c
        base = w * _SEG_W
        pltpu.sync_copy(svp_h.at[pl.ds(base, _SEG_W)], a0)
        pltpu.sync_copy(svp_h.at[pl.ds(_SEG_PAD + base, _SEG_W)], a1)
        pltpu.sync_copy(psp_h.at[pl.ds(base, _SEG_W)], p0)
        pltpu.sync_copy(psp_h.at[pl.ds(_SEG_PAD + base, _SEG_W)], p1)
        pltpu.sync_copy(pcp_h.at[pl.ds(base, _SEG_W)], c0)
        pltpu.sync_copy(pcp_h.at[pl.ds(_SEG_PAD + base, _SEG_W)], c1)

        def step(i, carry):
            sl = pl.ds(i * 16, 16)
            sv = a0[sl] + a1[sl]
            ps = p0[sl] + p1[sl]
            pc = c0[sl] + c1[sl]
            tb[sl] = sv - ps / jnp.maximum(pc, 1.0)
            return carry
        lax.fori_loop(0, _SEG_W // 16, step, 0)
        pltpu.sync_copy(tb, t_h.at[pl.ds(base, _SEG_W)])

    return k(svp, psp, pcp)


def _sc_gather_out(t, pi1, adv1):
    @functools.partial(
        pl.kernel, out_type=jax.ShapeDtypeStruct((_P_PAD,), jnp.float32),
        mesh=_sc_mesh(),
        scratch_types=[
            pltpu.VMEM((_PCH,), jnp.int32),
            pltpu.VMEM((_PCH,), jnp.float32),
            pltpu.VMEM((_PCH,), jnp.float32),
            pltpu.VMEM((_SEG_TILE,), jnp.float32),
            pltpu.VMEM_SHARED((_SEG_PAD,), jnp.float32),
            pltpu.SemaphoreType.DMA,
        ])
    def k(t_h, pi_h, adv_h, out_h, bpi, badv, btv, bstage, bts, sem):
        c = lax.axis_index("c")
        s = lax.axis_index("s")
        w = s * _NC + c
        sl = pl.ds(w * _PCH, _PCH)
        st_sl = pl.ds(s * _SEG_TILE, _SEG_TILE)
        pltpu.sync_copy(t_h.at[st_sl], bstage)
        pltpu.sync_copy(bstage, bts.at[st_sl])
        pltpu.sync_copy(pi_h.at[sl], bpi)
        pltpu.sync_copy(adv_h.at[sl], badv)
        plsc.subcore_barrier()
        pltpu.async_copy(bts.at[bpi], btv, sem).wait()

        def astep(i, carry):
            ssl = pl.ds(i * 16, 16)
            badv[ssl] = badv[ssl] + btv[ssl]
            return carry
        lax.fori_loop(0, _PCH // 16, astep, 0)
        pltpu.sync_copy(badv, out_h.at[sl])

    return k(t, pi1, adv1)


def kernel(x_movement, x_phase, mi_src, mi_dst, pi_dst, W1, b1, W2, b2):
    adv = _mlp(x_phase, W1, b1, W2, b2)
    p_pad = _P_PAD - N_PHASE
    pi1 = jnp.concatenate([pi_dst, jnp.full((p_pad,), _DUMMY_SEG, jnp.int32)])
    adv1 = jnp.concatenate([adv, jnp.zeros((p_pad,), jnp.float32)])
    psp, pcp = _sc_phase_accumulate(pi1, adv1)

    v_mov = _mlp(x_movement, W1, b1, W2, b2)
    v_mov = jnp.concatenate([v_mov, jnp.zeros((_V_PAD - N_MOV,), jnp.float32)])
    e_pad = _E_PAD - N_MOV
    msrc1 = jnp.concatenate([mi_src, jnp.zeros((e_pad,), jnp.int32)])
    mdst1 = jnp.concatenate([mi_dst, jnp.full((e_pad,), _DUMMY_SEG, jnp.int32)])
    svp = _sc_edge_accumulate(v_mov, msrc1, mdst1)

    t = _sc_combine(svp, psp, pcp)
    out_p = _sc_gather_out(t, pi1, adv1)
    return out_p[:N_PHASE], pi_dst


# speedup vs baseline: 57.5366x; 1.0287x over previous
import functools

import jax
import jax.numpy as jnp
from jax import lax
from jax.experimental import pallas as pl
from jax.experimental.pallas import tpu as pltpu
from jax.experimental.pallas import tpu_sc as plsc

N_MOV = 600000
N_PHASE = 400000
N_INT = 50000
D = 128
H = 256

_NC = 2
_NS = 16
_NW = _NC * _NS

_EW = 152
_PW = 104
_E_PAD = _NW * _EW * 128
_P_PAD = _NW * _PW * 128
_SEG_PAD = 50176
_SEG_TILE = _SEG_PAD // _NS
_SEG_W = _SEG_PAD // _NW
_DUMMY_SEG = 50100
_ECH = _EW * 128
_PCH = _PW * 128
_V_PAD = 600064
_VCH = _V_PAD // 32

_MLP_ROWS = 32768


def _mlp_block(x_ref, w1_ref, b1_ref, w2_ref, b2_ref, o_ref):
    x = x_ref[...].astype(jnp.bfloat16)
    w1 = w1_ref[...].astype(jnp.bfloat16)
    hT = lax.dot_general(w1, x, dimension_numbers=(((0,), (1,)), ((), ())),
                         preferred_element_type=jnp.float32)
    hT = jnp.maximum(hT + b1_ref[...], 0.0)
    o_ref[...] = jnp.sum(hT * w2_ref[...], axis=0) + b2_ref[0, 0]


def _mlp(x, W1, b1, W2, b2):
    n = x.shape[0]
    return pl.pallas_call(
        _mlp_block,
        grid=((n + _MLP_ROWS - 1) // _MLP_ROWS,),
        in_specs=[
            pl.BlockSpec((_MLP_ROWS, D), lambda i: (i, 0)),
            pl.BlockSpec((D, H), lambda i: (0, 0)),
            pl.BlockSpec((H, 1), lambda i: (0, 0)),
            pl.BlockSpec((H, 1), lambda i: (0, 0)),
            pl.BlockSpec((1, 1), lambda i: (0, 0)),
        ],
        out_specs=pl.BlockSpec((_MLP_ROWS,), lambda i: (i,)),
        out_shape=jax.ShapeDtypeStruct((n,), jnp.float32),
    )(x, W1, b1.reshape(H, 1), W2, b2.reshape(1, 1))


def _sc_mesh():
    return plsc.VectorSubcoreMesh(
        core_axis_name="c", subcore_axis_name="s",
        num_cores=_NC, num_subcores=_NS)


def _sc_phase_accumulate(pi1, adv1):
    out_type = (
        jax.ShapeDtypeStruct((_NC * _SEG_PAD,), jnp.float32),
        jax.ShapeDtypeStruct((_NC * _SEG_PAD,), jnp.float32),
    )

    @functools.partial(
        pl.kernel, out_type=out_type, mesh=_sc_mesh(),
        scratch_types=[
            pltpu.VMEM((_PCH,), jnp.int32),
            pltpu.VMEM((_PCH,), jnp.float32),
            pltpu.VMEM((_SEG_TILE,), jnp.float32),
            pltpu.VMEM_SHARED((_SEG_PAD,), jnp.float32),
            pltpu.VMEM_SHARED((_SEG_PAD,), jnp.float32),
        ])
    def k(pi_h, adv_h, psp_h, pcp_h, bpdst, bpval, bzero, acc_ps, acc_pc):
        c = lax.axis_index("c")
        s = lax.axis_index("s")
        w = s * _NC + c

        def zfill(i, carry):
            bzero[pl.ds(i * 16, 16)] = jnp.zeros((16,), jnp.float32)
            return carry
        lax.fori_loop(0, _SEG_TILE // 16, zfill, 0)

        tile_sl = pl.ds(s * _SEG_TILE, _SEG_TILE)
        pltpu.sync_copy(bzero, acc_ps.at[tile_sl])
        pltpu.sync_copy(bzero, acc_pc.at[tile_sl])
        pltpu.sync_copy(pi_h.at[pl.ds(w * _PCH, _PCH)], bpdst)
        pltpu.sync_copy(adv_h.at[pl.ds(w * _PCH, _PCH)], bpval)
        plsc.subcore_barrier()

        pltpu.sync_copy(bpval, acc_ps.at[bpdst], add=True)

        def ofill(i, carry):
            bpval[pl.ds(i * 16, 16)] = jnp.ones((16,), jnp.float32)
            return carry
        lax.fori_loop(0, _PCH // 16, ofill, 0)
        pltpu.sync_copy(bpval, acc_pc.at[bpdst], add=True)

        plsc.subcore_barrier()
        out_sl = pl.ds(c * _SEG_PAD + s * _SEG_TILE, _SEG_TILE)
        pltpu.sync_copy(acc_ps.at[tile_sl], bzero)
        pltpu.sync_copy(bzero, psp_h.at[out_sl])
        pltpu.sync_copy(acc_pc.at[tile_sl], bzero)
        pltpu.sync_copy(bzero, pcp_h.at[out_sl])

    return k(pi1, adv1)


def _sc_edge_accumulate(vmov, msrc1, mdst1):
    @functools.partial(
        pl.kernel,
        out_type=jax.ShapeDtypeStruct((_NC * _SEG_PAD,), jnp.float32),
        mesh=_sc_mesh(),
        scratch_types=[
            pltpu.VMEM((_ECH,), jnp.int32),
            pltpu.VMEM((_ECH,), jnp.int32),
            pltpu.VMEM((_ECH,), jnp.float32),
            pltpu.VMEM((_SEG_TILE,), jnp.float32),
            pltpu.VMEM_SHARED((_SEG_PAD,), jnp.float32),
            pltpu.VMEM_SHARED((_V_PAD,), jnp.float32),
            pltpu.SemaphoreType.DMA,
        ])
    def k(vmov_h, msrc_h, mdst_h, svp_h, bidx, bdst, bval, bzero,
          acc_sv, vms, sem):
        c = lax.axis_index("c")
        s = lax.axis_index("s")
        w = s * _NC + c

        def zfill(i, carry):
            bzero[pl.ds(i * 16, 16)] = jnp.zeros((16,), jnp.float32)
            return carry
        lax.fori_loop(0, _SEG_TILE // 16, zfill, 0)

        tile_sl = pl.ds(s * _SEG_TILE, _SEG_TILE)
        pltpu.sync_copy(bzero, acc_sv.at[tile_sl])

        for kk in range(2):
            st_sl = pl.ds(s * (2 * _VCH) + kk * _VCH, _VCH)
            pltpu.sync_copy(vmov_h.at[st_sl], bval.at[pl.ds(0, _VCH)])
            pltpu.sync_copy(bval.at[pl.ds(0, _VCH)], vms.at[st_sl])

        pltpu.sync_copy(msrc_h.at[pl.ds(w * _ECH, _ECH)], bidx)
        pltpu.sync_copy(mdst_h.at[pl.ds(w * _ECH, _ECH)], bdst)
        plsc.subcore_barrier()

        pltpu.async_copy(vms.at[bidx], bval, sem).wait()
        pltpu.sync_copy(bval, acc_sv.at[bdst], add=True)

        plsc.subcore_barrier()
        out_sl = pl.ds(c * _SEG_PAD + s * _SEG_TILE, _SEG_TILE)
        pltpu.sync_copy(acc_sv.at[tile_sl], bzero)
        pltpu.sync_copy(bzero, svp_h.at[out_sl])

    return k(vmov, msrc1, mdst1)


def _sc_combine(svp, psp, pcp):
    @functools.partial(
        pl.kernel, out_type=jax.ShapeDtypeStruct((_SEG_PAD,), jnp.float32),
        mesh=_sc_mesh(),
        scratch_types=[pltpu.VMEM((_SEG_W,), jnp.float32) for _ in range(7)])
    def k(svp_h, psp_h, pcp_h, t_h, a0, a1, p0, p1, c0, c1, tb):
        c = lax.axis_index("c")
        s = lax.axis_index("s")
        w = s * _NC + c
        base = w * _SEG_W
        pltpu.sync_copy(svp_h.at[pl.ds(base, _SEG_W)], a0)
        pltpu.sync_copy(svp_h.at[pl.ds(_SEG_PAD + base, _SEG_W)], a1)
        pltpu.sync_copy(psp_h.at[pl.ds(base, _SEG_W)], p0)
        pltpu.sync_copy(psp_h.at[pl.ds(_SEG_PAD + base, _SEG_W)], p1)
        pltpu.sync_copy(pcp_h.at[pl.ds(base, _SEG_W)], c0)
        pltpu.sync_copy(pcp_h.at[pl.ds(_SEG_PAD + base, _SEG_W)], c1)

        def step(i, carry):
            sl = pl.ds(i * 16, 16)
            sv = a0[sl] + a1[sl]
            ps = p0[sl] + p1[sl]
            pc = c0[sl] + c1[sl]
            tb[sl] = sv - ps / jnp.maximum(pc, 1.0)
            return carry
        lax.fori_loop(0, _SEG_W // 16, step, 0)
        pltpu.sync_copy(tb, t_h.at[pl.ds(base, _SEG_W)])

    return k(svp, psp, pcp)


def _sc_gather_out(t, pi1, adv1):
    @functools.partial(
        pl.kernel, out_type=jax.ShapeDtypeStruct((_P_PAD,), jnp.float32),
        mesh=_sc_mesh(),
        scratch_types=[
            pltpu.VMEM((_PCH,), jnp.int32),
            pltpu.VMEM((_PCH,), jnp.float32),
            pltpu.VMEM((_PCH,), jnp.float32),
            pltpu.VMEM((_SEG_TILE,), jnp.float32),
            pltpu.VMEM_SHARED((_SEG_PAD,), jnp.float32),
            pltpu.SemaphoreType.DMA,
        ])
    def k(t_h, pi_h, adv_h, out_h, bpi, badv, btv, bstage, bts, sem):
        c = lax.axis_index("c")
        s = lax.axis_index("s")
        w = s * _NC + c
        sl = pl.ds(w * _PCH, _PCH)
        st_sl = pl.ds(s * _SEG_TILE, _SEG_TILE)
        pltpu.sync_copy(t_h.at[st_sl], bstage)
        pltpu.sync_copy(bstage, bts.at[st_sl])
        pltpu.sync_copy(pi_h.at[sl], bpi)
        pltpu.sync_copy(adv_h.at[sl], badv)
        plsc.subcore_barrier()
        pltpu.async_copy(bts.at[bpi], btv, sem).wait()

        def astep(i, carry):
            ssl = pl.ds(i * 16, 16)
            badv[ssl] = badv[ssl] + btv[ssl]
            return carry
        lax.fori_loop(0, _PCH // 16, astep, 0)
        pltpu.sync_copy(badv, out_h.at[sl])

    return k(t, pi1, adv1)


def kernel(x_movement, x_phase, mi_src, mi_dst, pi_dst, W1, b1, W2, b2):
    adv = _mlp(x_phase, W1, b1, W2, b2)
    p_pad = _P_PAD - N_PHASE
    pi1 = jnp.concatenate([pi_dst, jnp.full((p_pad,), _DUMMY_SEG, jnp.int32)])
    adv1 = jnp.concatenate([adv, jnp.zeros((p_pad,), jnp.float32)])
    psp, pcp = _sc_phase_accumulate(pi1, adv1)

    v_mov = _mlp(x_movement, W1, b1, W2, b2)
    v_mov = jnp.concatenate([v_mov, jnp.zeros((_V_PAD - N_MOV,), jnp.float32)])
    e_pad = _E_PAD - N_MOV
    msrc1 = jnp.concatenate([mi_src, jnp.zeros((e_pad,), jnp.int32)])
    mdst1 = jnp.concatenate([mi_dst, jnp.full((e_pad,), _DUMMY_SEG, jnp.int32)])
    svp = _sc_edge_accumulate(v_mov, msrc1, mdst1)

    t = _sc_combine(svp, psp, pcp)
    out_p = _sc_gather_out(t, pi1, adv1)
    return out_p[:N_PHASE], pi_dst


# speedup vs baseline: 58.2754x; 1.0128x over previous
import functools

import jax
import jax.numpy as jnp
from jax import lax
from jax.experimental import pallas as pl
from jax.experimental.pallas import tpu as pltpu
from jax.experimental.pallas import tpu_sc as plsc

N_MOV = 600000
N_PHASE = 400000
N_INT = 50000
D = 128
H = 256

_NC = 2
_NS = 16
_NW = _NC * _NS

_EW = 152
_PW = 104
_E_PAD = _NW * _EW * 128
_P_PAD = _NW * _PW * 128
_SEG_PAD = 50176
_SEG_TILE = _SEG_PAD // _NS
_SEG_W = _SEG_PAD // _NW
_DUMMY_SEG = 50100
_ECH = _EW * 128
_PCH = _PW * 128
_V_PAD = 600064
_VCH = _V_PAD // 32

_MLP_ROWS = 32768


def _mlp_block(x_ref, w1_ref, b1_ref, w2_ref, b2_ref, o_ref):
    x = x_ref[...].astype(jnp.bfloat16)
    w1 = w1_ref[...].astype(jnp.bfloat16)
    hT = lax.dot_general(w1, x, dimension_numbers=(((0,), (1,)), ((), ())),
                         preferred_element_type=jnp.float32)
    hT = jnp.maximum(hT + b1_ref[...], 0.0)
    o_ref[...] = jnp.sum(hT * w2_ref[...], axis=0) + b2_ref[0, 0]


def _mlp(x, W1, b1, W2, b2):
    n = x.shape[0]
    return pl.pallas_call(
        _mlp_block,
        grid=((n + _MLP_ROWS - 1) // _MLP_ROWS,),
        in_specs=[
            pl.BlockSpec((_MLP_ROWS, D), lambda i: (i, 0)),
            pl.BlockSpec((D, H), lambda i: (0, 0)),
            pl.BlockSpec((H, 1), lambda i: (0, 0)),
            pl.BlockSpec((H, 1), lambda i: (0, 0)),
            pl.BlockSpec((1, 1), lambda i: (0, 0)),
        ],
        out_specs=pl.BlockSpec((_MLP_ROWS,), lambda i: (i,)),
        out_shape=jax.ShapeDtypeStruct((n,), jnp.float32),
    )(x, W1, b1.reshape(H, 1), W2, b2.reshape(1, 1))


def _sc_mesh():
    return plsc.VectorSubcoreMesh(
        core_axis_name="c", subcore_axis_name="s",
        num_cores=_NC, num_subcores=_NS)


def _sc_phase_accumulate(pi1, adv1):
    out_type = (
        jax.ShapeDtypeStruct((_NC * _SEG_PAD,), jnp.float32),
        jax.ShapeDtypeStruct((_NC * _SEG_PAD,), jnp.float32),
    )

    @functools.partial(
        pl.kernel, out_type=out_type, mesh=_sc_mesh(),
        scratch_types=[
            pltpu.VMEM((_PCH,), jnp.int32),
            pltpu.VMEM((_PCH,), jnp.float32),
            pltpu.VMEM((_SEG_TILE,), jnp.float32),
            pltpu.VMEM_SHARED((_SEG_PAD,), jnp.float32),
            pltpu.VMEM_SHARED((_SEG_PAD,), jnp.float32),
        ])
    def k(pi_h, adv_h, psp_h, pcp_h, bpdst, bpval, bzero, acc_ps, acc_pc):
        c = lax.axis_index("c")
        s = lax.axis_index("s")
        w = s * _NC + c

        def zfill(i, carry):
            bzero[pl.ds(i * 16, 16)] = jnp.zeros((16,), jnp.float32)
            return carry
        lax.fori_loop(0, _SEG_TILE // 16, zfill, 0)

        tile_sl = pl.ds(s * _SEG_TILE, _SEG_TILE)
        pltpu.sync_copy(bzero, acc_ps.at[tile_sl])
        pltpu.sync_copy(bzero, acc_pc.at[tile_sl])
        pltpu.sync_copy(pi_h.at[pl.ds(w * _PCH, _PCH)], bpdst)
        pltpu.sync_copy(adv_h.at[pl.ds(w * _PCH, _PCH)], bpval)
        plsc.subcore_barrier()

        pltpu.sync_copy(bpval, acc_ps.at[bpdst], add=True)

        def ofill(i, carry):
            bpval[pl.ds(i * 16, 16)] = jnp.ones((16,), jnp.float32)
            return carry
        lax.fori_loop(0, _PCH // 16, ofill, 0)
        pltpu.sync_copy(bpval, acc_pc.at[bpdst], add=True)

        plsc.subcore_barrier()
        out_sl = pl.ds(c * _SEG_PAD + s * _SEG_TILE, _SEG_TILE)
        pltpu.sync_copy(acc_ps.at[tile_sl], bzero)
        pltpu.sync_copy(bzero, psp_h.at[out_sl])
        pltpu.sync_copy(acc_pc.at[tile_sl], bzero)
        pltpu.sync_copy(bzero, pcp_h.at[out_sl])

    return k(pi1, adv1)


def _sc_edge_accumulate(vmov, msrc1, mdst1):
    @functools.partial(
        pl.kernel,
        out_type=jax.ShapeDtypeStruct((_NC * _SEG_PAD,), jnp.float32),
        mesh=_sc_mesh(),
        scratch_types=[
            pltpu.VMEM((_ECH,), jnp.int32),
            pltpu.VMEM((_ECH,), jnp.int32),
            pltpu.VMEM((_ECH,), jnp.float32),
            pltpu.VMEM((_SEG_TILE,), jnp.float32),
            pltpu.VMEM_SHARED((_SEG_PAD,), jnp.float32),
            pltpu.VMEM_SHARED((_V_PAD,), jnp.float32),
            pltpu.SemaphoreType.DMA,
        ])
    def k(vmov_h, msrc_h, mdst_h, svp_h, bidx, bdst, bval, bzero,
          acc_sv, vms, sem):
        c = lax.axis_index("c")
        s = lax.axis_index("s")
        w = s * _NC + c

        def zfill(i, carry):
            bzero[pl.ds(i * 16, 16)] = jnp.zeros((16,), jnp.float32)
            return carry
        lax.fori_loop(0, _SEG_TILE // 16, zfill, 0)

        tile_sl = pl.ds(s * _SEG_TILE, _SEG_TILE)
        pltpu.sync_copy(bzero, acc_sv.at[tile_sl])

        for kk in range(2):
            st_sl = pl.ds(s * (2 * _VCH) + kk * _VCH, _VCH)
            pltpu.sync_copy(vmov_h.at[st_sl], bval.at[pl.ds(0, _VCH)])
            pltpu.sync_copy(bval.at[pl.ds(0, _VCH)], vms.at[st_sl])

        pltpu.sync_copy(msrc_h.at[pl.ds(w * _ECH, _ECH)], bidx)
        pltpu.sync_copy(mdst_h.at[pl.ds(w * _ECH, _ECH)], bdst)
        plsc.subcore_barrier()

        pltpu.async_copy(vms.at[bidx], bval, sem).wait()
        pltpu.sync_copy(bval, acc_sv.at[bdst], add=True)

        plsc.subcore_barrier()
        out_sl = pl.ds(c * _SEG_PAD + s * _SEG_TILE, _SEG_TILE)
        pltpu.sync_copy(acc_sv.at[tile_sl], bzero)
        pltpu.sync_copy(bzero, svp_h.at[out_sl])

    return k(vmov, msrc1, mdst1)


def _sc_gather_out(svp, psp, pcp, pi1, adv1):
    @functools.partial(
        pl.kernel, out_type=jax.ShapeDtypeStruct((_P_PAD,), jnp.float32),
        mesh=_sc_mesh(),
        scratch_types=[
            pltpu.VMEM((_PCH,), jnp.int32),
            pltpu.VMEM((_PCH,), jnp.float32),
            pltpu.VMEM((_PCH,), jnp.float32),
            pltpu.VMEM((_SEG_TILE,), jnp.float32),
            pltpu.VMEM((_SEG_TILE,), jnp.float32),
            pltpu.VMEM((_SEG_TILE,), jnp.float32),
            pltpu.VMEM((_SEG_TILE,), jnp.float32),
            pltpu.VMEM((_SEG_TILE,), jnp.float32),
            pltpu.VMEM((_SEG_TILE,), jnp.float32),
            pltpu.VMEM((_SEG_TILE,), jnp.float32),
            pltpu.VMEM_SHARED((_SEG_PAD,), jnp.float32),
            pltpu.SemaphoreType.DMA,
        ])
    def k(svp_h, psp_h, pcp_h, pi_h, adv_h, out_h,
          bpi, badv, btv, a0, a1, p0, p1, q0, q1, tb, bts, sem):
        c = lax.axis_index("c")
        s = lax.axis_index("s")
        w = s * _NC + c
        sl = pl.ds(w * _PCH, _PCH)
        base = s * _SEG_TILE
        pltpu.sync_copy(svp_h.at[pl.ds(base, _SEG_TILE)], a0)
        pltpu.sync_copy(svp_h.at[pl.ds(_SEG_PAD + base, _SEG_TILE)], a1)
        pltpu.sync_copy(psp_h.at[pl.ds(base, _SEG_TILE)], p0)
        pltpu.sync_copy(psp_h.at[pl.ds(_SEG_PAD + base, _SEG_TILE)], p1)
        pltpu.sync_copy(pcp_h.at[pl.ds(base, _SEG_TILE)], q0)
        pltpu.sync_copy(pcp_h.at[pl.ds(_SEG_PAD + base, _SEG_TILE)], q1)

        def cstep(i, carry):
            ssl = pl.ds(i * 16, 16)
            sv = a0[ssl] + a1[ssl]
            ps = p0[ssl] + p1[ssl]
            pc = q0[ssl] + q1[ssl]
            tb[ssl] = sv - ps / jnp.maximum(pc, 1.0)
            return carry
        lax.fori_loop(0, _SEG_TILE // 16, cstep, 0)
        pltpu.sync_copy(tb, bts.at[pl.ds(base, _SEG_TILE)])
        pltpu.sync_copy(pi_h.at[sl], bpi)
        pltpu.sync_copy(adv_h.at[sl], badv)
        plsc.subcore_barrier()
        pltpu.async_copy(bts.at[bpi], btv, sem).wait()

        def astep(i, carry):
            ssl = pl.ds(i * 16, 16)
            badv[ssl] = badv[ssl] + btv[ssl]
            return carry
        lax.fori_loop(0, _PCH // 16, astep, 0)
        pltpu.sync_copy(badv, out_h.at[sl])

    return k(svp, psp, pcp, pi1, adv1)


def kernel(x_movement, x_phase, mi_src, mi_dst, pi_dst, W1, b1, W2, b2):
    adv = _mlp(x_phase, W1, b1, W2, b2)
    p_pad = _P_PAD - N_PHASE
    pi1 = jnp.concatenate([pi_dst, jnp.full((p_pad,), _DUMMY_SEG, jnp.int32)])
    adv1 = jnp.concatenate([adv, jnp.zeros((p_pad,), jnp.float32)])
    psp, pcp = _sc_phase_accumulate(pi1, adv1)

    v_mov = _mlp(x_movement, W1, b1, W2, b2)
    v_mov = jnp.concatenate([v_mov, jnp.zeros((_V_PAD - N_MOV,), jnp.float32)])
    e_pad = _E_PAD - N_MOV
    msrc1 = jnp.concatenate([mi_src, jnp.zeros((e_pad,), jnp.int32)])
    mdst1 = jnp.concatenate([mi_dst, jnp.full((e_pad,), _DUMMY_SEG, jnp.int32)])
    svp = _sc_edge_accumulate(v_mov, msrc1, mdst1)

    out_p = _sc_gather_out(svp, psp, pcp, pi1, adv1)
    return out_p[:N_PHASE], pi_dst
